# baseline (speedup 1.0000x reference)
import functools

import jax
import jax.numpy as jnp
from jax import lax
from jax.experimental import pallas as pl
from jax.experimental.pallas import tpu as pltpu
from jax.experimental.pallas import tpu_sc as plsc

T = 2048
D = 768
E = 16
NK = 2
P = 2 * T
BLK = 128
NPAD = P + E * BLK - 16
NPAD = 6144
NBLK = NPAD // BLK
L = 16


def _dot_nt(a, b):
    return lax.dot_general(a, b, (((1,), (1,)), ((), ())),
                           preferred_element_type=jnp.float32)


def _k1a_body(h_ref, wg_ref, lg_ref):
    lg_ref[...] = _dot_nt(h_ref[...], wg_ref[...])


def _k1a(hf, Wg, *, interpret=False):
    return pl.pallas_call(
        _k1a_body,
        grid=(1,),
        in_specs=[
            pl.BlockSpec((T, D), lambda t: (0, 0)),
            pl.BlockSpec((E, D), lambda t: (0, 0)),
        ],
        out_specs=pl.BlockSpec((T, E), lambda t: (0, 0)),
        out_shape=jax.ShapeDtypeStruct((T, E), jnp.float32),
        interpret=interpret,
    )(hf, Wg)


def _k1b_body(h_ref, w1s_ref, w2s_ref, w3s_ref, sh_ref):
    x = h_ref[...]
    z1 = _dot_nt(x, w1s_ref[...])
    z3 = _dot_nt(x, w3s_ref[...])
    act = z1 * jax.nn.sigmoid(z1) * z3
    sh_ref[...] = _dot_nt(act, w2s_ref[...])


def _k1b(hf, W1s, W2s, W3s, *, tt=256, interpret=False):
    grid = (T // tt,)
    return pl.pallas_call(
        _k1b_body,
        grid=grid,
        in_specs=[
            pl.BlockSpec((tt, D), lambda t: (t, 0)),
            pl.BlockSpec((D, D), lambda t: (0, 0)),
            pl.BlockSpec((D, D), lambda t: (0, 0)),
            pl.BlockSpec((D, D), lambda t: (0, 0)),
        ],
        out_specs=pl.BlockSpec((tt, D), lambda t: (t, 0)),
        out_shape=jax.ShapeDtypeStruct((T, D), jnp.float32),
        interpret=interpret,
    )(hf, W1s, W2s, W3s)


def _splat(v, lane):
    iota = lax.iota(jnp.int32, L)
    if v.dtype == jnp.int32:
        s = jnp.sum(jnp.where(iota == lane, v, 0))
    else:
        s = jnp.sum(jnp.where(iota == lane, v, 0.0))
    return jnp.full((L,), s, dtype=v.dtype)


def _k2_body(lg_hbm, src_hbm, slot_hbm, w_hbm, bexp_hbm, runof_hbm,
             runexp_hbm, nruns_hbm,
             lg_v, e0_v, e1_v, w0_v, w1_v, allids_v, cnts_v, c16_v,
             locsrc_v, locslot_v, acc_v, tmp_v, be_v, nx_v,
             nv_v, nv2_v, nr_v,
             ids_sp, cnt_sp, stsrc_sp, stslot_sp):
    c = lax.axis_index("c")
    s = lax.axis_index("s")
    on0 = c == 0
    iota = lax.iota(jnp.int32, L)

    @pl.when(on0)
    def _phase_a():
        pltpu.sync_copy(lg_hbm.at[pl.ds(s * (128 * E), 128 * E)], lg_v)

        def chunk(j, carry):
            base = j * L
            m1 = jnp.full((L,), -1e30, jnp.float32)
            m2 = jnp.full((L,), -1e30, jnp.float32)
            a1 = jnp.zeros((L,), jnp.int32)
            a2 = jnp.zeros((L,), jnp.int32)
            for e in range(E):
                ce = plsc.load_gather(lg_v, [(base + iota) * E + e])
                gt = ce > m1
                g2 = jnp.logical_and(jnp.logical_not(gt), ce > m2)
                m2n = jnp.where(gt, m1, jnp.where(g2, ce, m2))
                a2 = jnp.where(gt, a1, jnp.where(g2, e, a2))
                m2 = m2n
                a1 = jnp.where(gt, e, a1)
                m1 = jnp.where(gt, ce, m1)
            w0 = 1.0 / (1.0 + jnp.exp(m2 - m1))
            e0_v[pl.ds(base, L)] = a1
            e1_v[pl.ds(base, L)] = a2
            w0_v[pl.ds(base, L)] = w0
            w1_v[pl.ds(base, L)] = 1.0 - w0
            return carry

        lax.fori_loop(0, 128 // L, chunk, 0)
        pltpu.sync_copy(w0_v, w_hbm.at[pl.ds(s * 128, 128)])
        pltpu.sync_copy(w1_v, w_hbm.at[pl.ds(T + s * 128, 128)])
        pltpu.sync_copy(e0_v, ids_sp.at[pl.ds(s * 128, 128)])
        pltpu.sync_copy(e1_v, ids_sp.at[pl.ds(T + s * 128, 128)])

    plsc.subcore_barrier()

    @pl.when(on0)
    def _phase_b():
        pltpu.sync_copy(ids_sp, allids_v)

        def cb(i, cnt):
            v = allids_v[pl.ds(i * L, L)]
            return cnt + (v == s).astype(jnp.int32)

        cnt = lax.fori_loop(0, P // L, cb, jnp.zeros((L,), jnp.int32))
        tot = jnp.sum(cnt)
        c16_v[...] = jnp.full((L,), tot, jnp.int32)
        pltpu.sync_copy(c16_v, cnt_sp.at[s])

    plsc.subcore_barrier()

    @pl.when(on0)
    def _phase_c():
        pltpu.sync_copy(cnt_sp, cnts_v)
        counts = plsc.load_gather(cnts_v, [iota, iota])
        padded = ((counts + (BLK - 1)) >> 7) << 7
        cs = plsc.cumsum(padded)
        start = cs - padded

        def z1(i, carry):
            locsrc_v[pl.ds(i * L, L)] = jnp.zeros((L,), jnp.int32)
            return carry

        def z2(i, carry):
            locslot_v[pl.ds(i * L, L)] = jnp.zeros((L,), jnp.int32)
            return carry

        lax.fori_loop(0, NPAD // L, z1, 0)
        lax.fori_loop(0, P // L, z2, 0)

        start_s = _splat(start, s)

        def ce(i, rank):
            v = allids_v[pl.ds(i * L, L)]
            m = v == s
            mi = m.astype(jnp.int32)
            pcs = plsc.cumsum(mi)
            slots = rank + pcs - mi
            pairpos = i * L + iota
            tok = jnp.bitwise_and(pairpos, T - 1)
            plsc.store_scatter(locslot_v, [pairpos], slots, mask=m)
            plsc.store_scatter(locsrc_v, [slots], tok + 1, mask=m)
            return rank + _splat(pcs, L - 1)

        lax.fori_loop(0, P // L, ce, start_s)

        @pl.when(s == 0)
        def _bexp():
            startblk = start >> 7
            ne = (padded > 0).astype(jnp.int32)
            rank = plsc.cumsum(ne) - ne
            nv_v[...] = jnp.zeros((L,), jnp.int32)
            plsc.store_scatter(nv_v, [rank], iota, mask=padded > 0)
            nv2_v[...] = rank
            nr_v[...] = jnp.full((L,), jnp.sum(ne), jnp.int32)
            for cc in range(NBLK // L):
                bvec = iota + cc * L
                acc = jnp.zeros((L,), jnp.int32)
                for e in range(E):
                    sb = _splat(startblk, e)
                    pe = _splat(padded, e)
                    cond = jnp.logical_and(sb <= bvec, pe > 0)
                    acc = jnp.where(cond, e, acc)
                be_v[pl.ds(cc * L, L)] = acc
                nx_v[pl.ds(cc * L, L)] = plsc.load_gather(nv2_v, [acc])
            pltpu.sync_copy(be_v, bexp_hbm)
            pltpu.sync_copy(nx_v, runof_hbm)
            pltpu.sync_copy(nv_v, runexp_hbm)
            pltpu.sync_copy(nr_v, nruns_hbm)

        pltpu.sync_copy(locsrc_v, stsrc_sp.at[s])
        pltpu.sync_copy(locslot_v, stslot_sp.at[s])

    plsc.subcore_barrier()

    SRCW = NPAD // L
    SLTW = P // L

    @pl.when(on0)
    def _phase_d():
        pltpu.sync_copy(stsrc_sp.at[0, pl.ds(s * SRCW, SRCW)], acc_v.at[pl.ds(0, SRCW)])
        for r in range(1, L):
            pltpu.sync_copy(stsrc_sp.at[r, pl.ds(s * SRCW, SRCW)], tmp_v.at[pl.ds(0, SRCW)])
            for j in range(SRCW // L):
                acc_v[pl.ds(j * L, L)] = acc_v[pl.ds(j * L, L)] + tmp_v[pl.ds(j * L, L)]
        for j in range(SRCW // L):
            a = acc_v[pl.ds(j * L, L)]
            ramp = jnp.bitwise_and(s * SRCW + j * L + iota, T - 1)
            acc_v[pl.ds(j * L, L)] = jnp.where(a > 0, a - 1, ramp)
        pltpu.sync_copy(acc_v.at[pl.ds(0, SRCW)], src_hbm.at[pl.ds(s * SRCW, SRCW)])

        pltpu.sync_copy(stslot_sp.at[0, pl.ds(s * SLTW, SLTW)], acc_v.at[pl.ds(0, SLTW)])
        for r in range(1, L):
            pltpu.sync_copy(stslot_sp.at[r, pl.ds(s * SLTW, SLTW)], tmp_v.at[pl.ds(0, SLTW)])
            for j in range(SLTW // L):
                acc_v[pl.ds(j * L, L)] = acc_v[pl.ds(j * L, L)] + tmp_v[pl.ds(j * L, L)]
        pltpu.sync_copy(acc_v.at[pl.ds(0, SLTW)], slot_hbm.at[pl.ds(s * SLTW, SLTW)])


def _k2(logits_flat):
    mesh = plsc.VectorSubcoreMesh(core_axis_name="c", subcore_axis_name="s")
    f = pl.kernel(
        _k2_body,
        compiler_params=pltpu.CompilerParams(needs_layout_passes=False),
        out_type=[
            jax.ShapeDtypeStruct((NPAD,), jnp.int32),
            jax.ShapeDtypeStruct((P,), jnp.int32),
            jax.ShapeDtypeStruct((P,), jnp.float32),
            jax.ShapeDtypeStruct((NBLK,), jnp.int32),
            jax.ShapeDtypeStruct((NBLK,), jnp.int32),
            jax.ShapeDtypeStruct((L,), jnp.int32),
            jax.ShapeDtypeStruct((L,), jnp.int32),
        ],
        mesh=mesh,
        scratch_types=[
            pltpu.VMEM((128 * E,), jnp.float32),
            pltpu.VMEM((128,), jnp.int32),
            pltpu.VMEM((128,), jnp.int32),
            pltpu.VMEM((128,), jnp.float32),
            pltpu.VMEM((128,), jnp.float32),
            pltpu.VMEM((P,), jnp.int32),
            pltpu.VMEM((L, L), jnp.int32),
            pltpu.VMEM((L,), jnp.int32),
            pltpu.VMEM((NPAD,), jnp.int32),
            pltpu.VMEM((P,), jnp.int32),
            pltpu.VMEM((NPAD // L,), jnp.int32),
            pltpu.VMEM((NPAD // L,), jnp.int32),
            pltpu.VMEM((NBLK,), jnp.int32),
            pltpu.VMEM((NBLK,), jnp.int32),
            pltpu.VMEM((L,), jnp.int32),
            pltpu.VMEM((L,), jnp.int32),
            pltpu.VMEM((L,), jnp.int32),
            pltpu.VMEM_SHARED((P,), jnp.int32),
            pltpu.VMEM_SHARED((L, L), jnp.int32),
            pltpu.VMEM_SHARED((L, NPAD), jnp.int32),
            pltpu.VMEM_SHARED((L, P), jnp.int32),
        ],
    )
    return f(logits_flat)


GCH = 32
GNCH = 6
GNBUF = 4


def _k2b_body(src_hbm, hf_hbm, xs_hbm, idx_v, r0, r1, r2, r3,
              g0, g1, g2, g3, w0, w1, w2, w3):
    c = lax.axis_index("c")
    s = lax.axis_index("s")
    wid = s * 2 + c
    per = NPAD // 32
    base = wid * per
    bufs = (r0, r1, r2, r3)
    gsems = (g0, g1, g2, g3)
    wsems = (w0, w1, w2, w3)
    pltpu.sync_copy(src_hbm.at[pl.ds(base, per)], idx_v)
    gcp = [None] * GNCH
    wcp = [None] * GNCH
    for j in range(GNCH):
        b = j % GNBUF
        if j >= GNBUF:
            wcp[j - GNBUF].wait()
        gcp[j] = pltpu.async_copy(
            hf_hbm.at[idx_v.at[pl.ds(j * GCH, GCH)]], bufs[b], gsems[b])
        if j >= 1:
            bp = (j - 1) % GNBUF
            gcp[j - 1].wait()
            wcp[j - 1] = pltpu.async_copy(
                bufs[bp], xs_hbm.at[pl.ds(base + (j - 1) * GCH, GCH), :],
                wsems[bp])
    gcp[GNCH - 1].wait()
    wcp[GNCH - 1] = pltpu.async_copy(
        bufs[(GNCH - 1) % GNBUF],
        xs_hbm.at[pl.ds(base + (GNCH - 1) * GCH, GCH), :],
        wsems[(GNCH - 1) % GNBUF])
    for j in range(GNCH - GNBUF, GNCH):
        if j >= 0 and wcp[j] is not None:
            wcp[j].wait()


def _k2b(src_idx, hf):
    mesh = plsc.VectorSubcoreMesh(core_axis_name="c", subcore_axis_name="s")
    f = pl.kernel(
        _k2b_body,
        compiler_params=pltpu.CompilerParams(needs_layout_passes=False),
        out_type=jax.ShapeDtypeStruct((NPAD, D), jnp.float32),
        mesh=mesh,
        scratch_types=[
            pltpu.VMEM((NPAD // 32,), jnp.int32),
            pltpu.VMEM((GCH, D), jnp.float32),
            pltpu.VMEM((GCH, D), jnp.float32),
            pltpu.VMEM((GCH, D), jnp.float32),
            pltpu.VMEM((GCH, D), jnp.float32),
            pltpu.SemaphoreType.DMA,
            pltpu.SemaphoreType.DMA,
            pltpu.SemaphoreType.DMA,
            pltpu.SemaphoreType.DMA,
            pltpu.SemaphoreType.DMA,
            pltpu.SemaphoreType.DMA,
            pltpu.SemaphoreType.DMA,
            pltpu.SemaphoreType.DMA,
        ],
    )
    return f(src_idx, hf)


NSLOT = 5


def _k3_body(be_ref, runof_ref, runexp_ref, nruns_ref, x_ref,
             w1_hbm, w2_hbm, w3_hbm, o_ref, w1b, w2b, w3b, s1, s2, s3):
    b = pl.program_id(0)
    nruns = nruns_ref[0]

    def fetch(e, slot):
        pltpu.async_copy(w1_hbm.at[e], w1b.at[slot], s1.at[slot])
        pltpu.async_copy(w2_hbm.at[e], w2b.at[slot], s2.at[slot])
        pltpu.async_copy(w3_hbm.at[e], w3b.at[slot], s3.at[slot])

    def wait(slot):
        pltpu.make_async_copy(w1_hbm.at[0], w1b.at[slot], s1.at[slot]).wait()
        pltpu.make_async_copy(w2_hbm.at[0], w2b.at[slot], s2.at[slot]).wait()
        pltpu.make_async_copy(w3_hbm.at[0], w3b.at[slot], s3.at[slot]).wait()

    rc = runof_ref[b]

    @pl.when(b == 0)
    def _init():
        for r in range(NSLOT):
            @pl.when(r < nruns)
            def _():
                fetch(runexp_ref[r], r)

        wait(0)

    changed = jnp.logical_and(b > 0, rc != runof_ref[jnp.maximum(b - 1, 0)])

    @pl.when(changed)
    def _advance():
        pr = rc + NSLOT - 1

        @pl.when(pr < nruns)
        def _():
            fetch(runexp_ref[pr], lax.rem(pr, NSLOT))

        wait(lax.rem(rc, NSLOT))

    slot = lax.rem(rc, NSLOT)
    x = x_ref[...]
    z1 = _dot_nt(x, w1b[slot])
    z3 = _dot_nt(x, w3b[slot])
    act = z1 * jax.nn.sigmoid(z1) * z3
    o_ref[...] = _dot_nt(act, w2b[slot])


def _k3(block_expert, runof, runexp, nruns, xs, W1, W2, W3):
    grid_spec = pltpu.PrefetchScalarGridSpec(
        num_scalar_prefetch=4,
        grid=(NBLK,),
        in_specs=[
            pl.BlockSpec((BLK, D), lambda b, be, ro, re, nr: (b, 0)),
            pl.BlockSpec(memory_space=pl.ANY),
            pl.BlockSpec(memory_space=pl.ANY),
            pl.BlockSpec(memory_space=pl.ANY),
        ],
        out_specs=pl.BlockSpec((BLK, D), lambda b, be, ro, re, nr: (b, 0)),
        scratch_shapes=[
            pltpu.VMEM((NSLOT, D, D), jnp.float32),
            pltpu.VMEM((NSLOT, D, D), jnp.float32),
            pltpu.VMEM((NSLOT, D, D), jnp.float32),
            pltpu.SemaphoreType.DMA((NSLOT,)),
            pltpu.SemaphoreType.DMA((NSLOT,)),
            pltpu.SemaphoreType.DMA((NSLOT,)),
        ],
    )
    return pl.pallas_call(
        _k3_body,
        grid_spec=grid_spec,
        out_shape=jax.ShapeDtypeStruct((NPAD, D), jnp.float32),
    )(block_expert, runof, runexp, nruns, xs, W1, W2, W3)


CCH = 32


def _k4_body(outs_hbm, sh_hbm, slot_hbm, w_hbm, y_hbm,
             idx0_v, idx1_v, w0_v, w1_v, r0_v, r1_v, shv_v, y_v,
             sem, sem2, sem3, ysem):
    c = lax.axis_index("c")
    s = lax.axis_index("s")
    wid = s * 2 + c
    per = T // 32
    iota = lax.iota(jnp.int32, L)
    ycp = None
    for jc in range(per // CCH):
        t0 = wid * per + jc * CCH
        pltpu.sync_copy(slot_hbm.at[pl.ds(t0, CCH)], idx0_v)
        pltpu.sync_copy(slot_hbm.at[pl.ds(T + t0, CCH)], idx1_v)
        pltpu.sync_copy(w_hbm.at[pl.ds(t0, CCH)], w0_v)
        pltpu.sync_copy(w_hbm.at[pl.ds(T + t0, CCH)], w1_v)
        cp0 = pltpu.async_copy(outs_hbm.at[idx0_v], r0_v, sem)
        cp1 = pltpu.async_copy(outs_hbm.at[idx1_v], r1_v, sem2)
        cps = pltpu.async_copy(sh_hbm.at[pl.ds(t0, CCH), :], shv_v, sem3)
        cp0.wait()
        cp1.wait()
        cps.wait()
        if ycp is not None:
            ycp.wait()

        def tok(i, carry):
            g = i >> 4
            lane = jnp.bitwise_and(i, L - 1)
            w0g = w0_v[pl.ds(g * L, L)]
            w1g = w1_v[pl.ds(g * L, L)]
            w0s = jnp.full((L,), jnp.sum(jnp.where(iota == lane, w0g, 0.0)), jnp.float32)
            w1s = jnp.full((L,), jnp.sum(jnp.where(iota == lane, w1g, 0.0)), jnp.float32)

            def col(jj, carry2):
                sl = pl.ds(jj * L, L)
                y_v[i, sl] = (r0_v[i, sl] * w0s + r1_v[i, sl] * w1s
                              + shv_v[i, sl])
                return carry2

            lax.fori_loop(0, D // L, col, 0)
            return carry

        lax.fori_loop(0, CCH, tok, 0)
        ycp = pltpu.async_copy(y_v, y_hbm.at[pl.ds(t0, CCH), :], ysem)
    ycp.wait()


def _k4(outs, shared_y, slot_flat, w_flat):
    mesh = plsc.VectorSubcoreMesh(core_axis_name="c", subcore_axis_name="s")
    f = pl.kernel(
        _k4_body,
        compiler_params=pltpu.CompilerParams(needs_layout_passes=False),
        out_type=jax.ShapeDtypeStruct((T, D), jnp.float32),
        mesh=mesh,
        scratch_types=[
            pltpu.VMEM((CCH,), jnp.int32),
            pltpu.VMEM((CCH,), jnp.int32),
            pltpu.VMEM((CCH,), jnp.float32),
            pltpu.VMEM((CCH,), jnp.float32),
            pltpu.VMEM((CCH, D), jnp.float32),
            pltpu.VMEM((CCH, D), jnp.float32),
            pltpu.VMEM((CCH, D), jnp.float32),
            pltpu.VMEM((CCH, D), jnp.float32),
            pltpu.SemaphoreType.DMA,
            pltpu.SemaphoreType.DMA,
            pltpu.SemaphoreType.DMA,
            pltpu.SemaphoreType.DMA,
        ],
    )
    return f(outs, shared_y, slot_flat, w_flat)


def kernel(h, Wg, W1, W2, W3, W1s, W2s, W3s):
    b, s, d = h.shape
    hf = h.reshape(T, D)
    logits = _k1a(hf, Wg)
    shared_y = _k1b(hf, W1s, W2s, W3s)
    src_idx, slot_flat, w_flat, block_expert, runof, runexp, nruns = _k2(
        logits.reshape(-1))
    xs = _k2b(src_idx, hf)
    outs = _k3(block_expert, runof, runexp, nruns, xs, W1, W2, W3)
    y = _k4(outs, shared_y, slot_flat, w_flat)
    return y.reshape(b, s, d)


# speedup vs baseline: 1.0020x; 1.0020x over previous
import functools

import jax
import jax.numpy as jnp
from jax import lax
from jax.experimental import pallas as pl
from jax.experimental.pallas import tpu as pltpu
from jax.experimental.pallas import tpu_sc as plsc

T = 2048
D = 768
E = 16
NK = 2
P = 2 * T
BLK = 128
NPAD = P + E * BLK - 16
NPAD = 6144
NBLK = NPAD // BLK
L = 16


def _dot_nt(a, b):
    return lax.dot_general(a, b, (((1,), (1,)), ((), ())),
                           precision=lax.Precision.DEFAULT,
                           preferred_element_type=jnp.float32)


def _k1a_body(h_ref, wg_ref, lg_ref):
    lg_ref[...] = lax.dot_general(
        h_ref[...], wg_ref[...], (((1,), (1,)), ((), ())),
        precision=lax.Precision.HIGHEST,
        preferred_element_type=jnp.float32)


def _k1a(hf, Wg, *, interpret=False):
    return pl.pallas_call(
        _k1a_body,
        grid=(1,),
        in_specs=[
            pl.BlockSpec((T, D), lambda t: (0, 0)),
            pl.BlockSpec((E, D), lambda t: (0, 0)),
        ],
        out_specs=pl.BlockSpec((T, E), lambda t: (0, 0)),
        out_shape=jax.ShapeDtypeStruct((T, E), jnp.float32),
        interpret=interpret,
    )(hf, Wg)


def _k1b_body(h_ref, w1s_ref, w2s_ref, w3s_ref, sh_ref):
    x = h_ref[...]
    z1 = _dot_nt(x, w1s_ref[...])
    z3 = _dot_nt(x, w3s_ref[...])
    act = z1 * jax.nn.sigmoid(z1) * z3
    sh_ref[...] = _dot_nt(act, w2s_ref[...])


def _k1b(hf, W1s, W2s, W3s, *, tt=256, interpret=False):
    grid = (T // tt,)
    return pl.pallas_call(
        _k1b_body,
        grid=grid,
        in_specs=[
            pl.BlockSpec((tt, D), lambda t: (t, 0)),
            pl.BlockSpec((D, D), lambda t: (0, 0)),
            pl.BlockSpec((D, D), lambda t: (0, 0)),
            pl.BlockSpec((D, D), lambda t: (0, 0)),
        ],
        out_specs=pl.BlockSpec((tt, D), lambda t: (t, 0)),
        out_shape=jax.ShapeDtypeStruct((T, D), jnp.float32),
        interpret=interpret,
    )(hf, W1s, W2s, W3s)


def _splat(v, lane):
    iota = lax.iota(jnp.int32, L)
    if v.dtype == jnp.int32:
        s = jnp.sum(jnp.where(iota == lane, v, 0))
    else:
        s = jnp.sum(jnp.where(iota == lane, v, 0.0))
    return jnp.full((L,), s, dtype=v.dtype)


def _k2_body(lg_hbm, src_hbm, slot_hbm, w_hbm, bexp_hbm, runof_hbm,
             runexp_hbm, nruns_hbm,
             lg_v, e0_v, e1_v, w0_v, w1_v, allids_v, cnts_v, c16_v,
             locsrc_v, locslot_v, acc_v, tmp_v, be_v, nx_v,
             nv_v, nv2_v, nr_v,
             ids_sp, cnt_sp, stsrc_sp, stslot_sp):
    c = lax.axis_index("c")
    s = lax.axis_index("s")
    on0 = c == 0
    iota = lax.iota(jnp.int32, L)

    @pl.when(on0)
    def _phase_a():
        pltpu.sync_copy(lg_hbm.at[pl.ds(s * (128 * E), 128 * E)], lg_v)

        def chunk(j, carry):
            base = j * L
            m1 = jnp.full((L,), -1e30, jnp.float32)
            m2 = jnp.full((L,), -1e30, jnp.float32)
            a1 = jnp.zeros((L,), jnp.int32)
            a2 = jnp.zeros((L,), jnp.int32)
            for e in range(E):
                ce = plsc.load_gather(lg_v, [(base + iota) * E + e])
                gt = ce > m1
                g2 = jnp.logical_and(jnp.logical_not(gt), ce > m2)
                m2n = jnp.where(gt, m1, jnp.where(g2, ce, m2))
                a2 = jnp.where(gt, a1, jnp.where(g2, e, a2))
                m2 = m2n
                a1 = jnp.where(gt, e, a1)
                m1 = jnp.where(gt, ce, m1)
            w0 = 1.0 / (1.0 + jnp.exp(m2 - m1))
            e0_v[pl.ds(base, L)] = a1
            e1_v[pl.ds(base, L)] = a2
            w0_v[pl.ds(base, L)] = w0
            w1_v[pl.ds(base, L)] = 1.0 - w0
            return carry

        lax.fori_loop(0, 128 // L, chunk, 0)
        pltpu.sync_copy(w0_v, w_hbm.at[pl.ds(s * 128, 128)])
        pltpu.sync_copy(w1_v, w_hbm.at[pl.ds(T + s * 128, 128)])
        pltpu.sync_copy(e0_v, ids_sp.at[pl.ds(s * 128, 128)])
        pltpu.sync_copy(e1_v, ids_sp.at[pl.ds(T + s * 128, 128)])

    plsc.subcore_barrier()

    @pl.when(on0)
    def _phase_b():
        pltpu.sync_copy(ids_sp, allids_v)

        def cb(i, cnt):
            v = allids_v[pl.ds(i * L, L)]
            return cnt + (v == s).astype(jnp.int32)

        cnt = lax.fori_loop(0, P // L, cb, jnp.zeros((L,), jnp.int32))
        tot = jnp.sum(cnt)
        c16_v[...] = jnp.full((L,), tot, jnp.int32)
        pltpu.sync_copy(c16_v, cnt_sp.at[s])

    plsc.subcore_barrier()

    @pl.when(on0)
    def _phase_c():
        pltpu.sync_copy(cnt_sp, cnts_v)
        counts = plsc.load_gather(cnts_v, [iota, iota])
        padded = ((counts + (BLK - 1)) >> 7) << 7
        cs = plsc.cumsum(padded)
        start = cs - padded

        def z1(i, carry):
            locsrc_v[pl.ds(i * L, L)] = jnp.zeros((L,), jnp.int32)
            return carry

        def z2(i, carry):
            locslot_v[pl.ds(i * L, L)] = jnp.zeros((L,), jnp.int32)
            return carry

        lax.fori_loop(0, NPAD // L, z1, 0)
        lax.fori_loop(0, P // L, z2, 0)

        start_s = _splat(start, s)

        def ce(i, rank):
            v = allids_v[pl.ds(i * L, L)]
            m = v == s
            mi = m.astype(jnp.int32)
            pcs = plsc.cumsum(mi)
            slots = rank + pcs - mi
            pairpos = i * L + iota
            tok = jnp.bitwise_and(pairpos, T - 1)
            plsc.store_scatter(locslot_v, [pairpos], slots, mask=m)
            plsc.store_scatter(locsrc_v, [slots], tok + 1, mask=m)
            return rank + _splat(pcs, L - 1)

        lax.fori_loop(0, P // L, ce, start_s)

        @pl.when(s == 0)
        def _bexp():
            startblk = start >> 7
            ne = (padded > 0).astype(jnp.int32)
            rank = plsc.cumsum(ne) - ne
            nv_v[...] = jnp.zeros((L,), jnp.int32)
            plsc.store_scatter(nv_v, [rank], iota, mask=padded > 0)
            nv2_v[...] = rank
            nr_v[...] = jnp.full((L,), jnp.sum(ne), jnp.int32)
            for cc in range(NBLK // L):
                bvec = iota + cc * L
                acc = jnp.zeros((L,), jnp.int32)
                for e in range(E):
                    sb = _splat(startblk, e)
                    pe = _splat(padded, e)
                    cond = jnp.logical_and(sb <= bvec, pe > 0)
                    acc = jnp.where(cond, e, acc)
                be_v[pl.ds(cc * L, L)] = acc
                nx_v[pl.ds(cc * L, L)] = plsc.load_gather(nv2_v, [acc])
            pltpu.sync_copy(be_v, bexp_hbm)
            pltpu.sync_copy(nx_v, runof_hbm)
            pltpu.sync_copy(nv_v, runexp_hbm)
            pltpu.sync_copy(nr_v, nruns_hbm)

        pltpu.sync_copy(locsrc_v, stsrc_sp.at[s])
        pltpu.sync_copy(locslot_v, stslot_sp.at[s])

    plsc.subcore_barrier()

    SRCW = NPAD // L
    SLTW = P // L

    @pl.when(on0)
    def _phase_d():
        pltpu.sync_copy(stsrc_sp.at[0, pl.ds(s * SRCW, SRCW)], acc_v.at[pl.ds(0, SRCW)])
        for r in range(1, L):
            pltpu.sync_copy(stsrc_sp.at[r, pl.ds(s * SRCW, SRCW)], tmp_v.at[pl.ds(0, SRCW)])
            for j in range(SRCW // L):
                acc_v[pl.ds(j * L, L)] = acc_v[pl.ds(j * L, L)] + tmp_v[pl.ds(j * L, L)]
        for j in range(SRCW // L):
            a = acc_v[pl.ds(j * L, L)]
            ramp = jnp.bitwise_and(s * SRCW + j * L + iota, T - 1)
            acc_v[pl.ds(j * L, L)] = jnp.where(a > 0, a - 1, ramp)
        pltpu.sync_copy(acc_v.at[pl.ds(0, SRCW)], src_hbm.at[pl.ds(s * SRCW, SRCW)])

        pltpu.sync_copy(stslot_sp.at[0, pl.ds(s * SLTW, SLTW)], acc_v.at[pl.ds(0, SLTW)])
        for r in range(1, L):
            pltpu.sync_copy(stslot_sp.at[r, pl.ds(s * SLTW, SLTW)], tmp_v.at[pl.ds(0, SLTW)])
            for j in range(SLTW // L):
                acc_v[pl.ds(j * L, L)] = acc_v[pl.ds(j * L, L)] + tmp_v[pl.ds(j * L, L)]
        pltpu.sync_copy(acc_v.at[pl.ds(0, SLTW)], slot_hbm.at[pl.ds(s * SLTW, SLTW)])


def _k2(logits_flat):
    mesh = plsc.VectorSubcoreMesh(core_axis_name="c", subcore_axis_name="s")
    f = pl.kernel(
        _k2_body,
        compiler_params=pltpu.CompilerParams(needs_layout_passes=False),
        out_type=[
            jax.ShapeDtypeStruct((NPAD,), jnp.int32),
            jax.ShapeDtypeStruct((P,), jnp.int32),
            jax.ShapeDtypeStruct((P,), jnp.float32),
            jax.ShapeDtypeStruct((NBLK,), jnp.int32),
            jax.ShapeDtypeStruct((NBLK,), jnp.int32),
            jax.ShapeDtypeStruct((L,), jnp.int32),
            jax.ShapeDtypeStruct((L,), jnp.int32),
        ],
        mesh=mesh,
        scratch_types=[
            pltpu.VMEM((128 * E,), jnp.float32),
            pltpu.VMEM((128,), jnp.int32),
            pltpu.VMEM((128,), jnp.int32),
            pltpu.VMEM((128,), jnp.float32),
            pltpu.VMEM((128,), jnp.float32),
            pltpu.VMEM((P,), jnp.int32),
            pltpu.VMEM((L, L), jnp.int32),
            pltpu.VMEM((L,), jnp.int32),
            pltpu.VMEM((NPAD,), jnp.int32),
            pltpu.VMEM((P,), jnp.int32),
            pltpu.VMEM((NPAD // L,), jnp.int32),
            pltpu.VMEM((NPAD // L,), jnp.int32),
            pltpu.VMEM((NBLK,), jnp.int32),
            pltpu.VMEM((NBLK,), jnp.int32),
            pltpu.VMEM((L,), jnp.int32),
            pltpu.VMEM((L,), jnp.int32),
            pltpu.VMEM((L,), jnp.int32),
            pltpu.VMEM_SHARED((P,), jnp.int32),
            pltpu.VMEM_SHARED((L, L), jnp.int32),
            pltpu.VMEM_SHARED((L, NPAD), jnp.int32),
            pltpu.VMEM_SHARED((L, P), jnp.int32),
        ],
    )
    return f(logits_flat)


GCH = 32
GNCH = 6
GNBUF = 4


def _k2b_body(src_hbm, hf_hbm, xs_hbm, idx_v, r0, r1, r2, r3,
              g0, g1, g2, g3, w0, w1, w2, w3):
    c = lax.axis_index("c")
    s = lax.axis_index("s")
    wid = s * 2 + c
    per = NPAD // 32
    base = wid * per
    bufs = (r0, r1, r2, r3)
    gsems = (g0, g1, g2, g3)
    wsems = (w0, w1, w2, w3)
    pltpu.sync_copy(src_hbm.at[pl.ds(base, per)], idx_v)
    gcp = [None] * GNCH
    wcp = [None] * GNCH
    for j in range(GNCH):
        b = j % GNBUF
        if j >= GNBUF:
            wcp[j - GNBUF].wait()
        gcp[j] = pltpu.async_copy(
            hf_hbm.at[idx_v.at[pl.ds(j * GCH, GCH)]], bufs[b], gsems[b])
        if j >= 1:
            bp = (j - 1) % GNBUF
            gcp[j - 1].wait()
            wcp[j - 1] = pltpu.async_copy(
                bufs[bp], xs_hbm.at[pl.ds(base + (j - 1) * GCH, GCH), :],
                wsems[bp])
    gcp[GNCH - 1].wait()
    wcp[GNCH - 1] = pltpu.async_copy(
        bufs[(GNCH - 1) % GNBUF],
        xs_hbm.at[pl.ds(base + (GNCH - 1) * GCH, GCH), :],
        wsems[(GNCH - 1) % GNBUF])
    for j in range(GNCH - GNBUF, GNCH):
        if j >= 0 and wcp[j] is not None:
            wcp[j].wait()


def _k2b(src_idx, hf):
    mesh = plsc.VectorSubcoreMesh(core_axis_name="c", subcore_axis_name="s")
    f = pl.kernel(
        _k2b_body,
        compiler_params=pltpu.CompilerParams(needs_layout_passes=False),
        out_type=jax.ShapeDtypeStruct((NPAD, D), jnp.float32),
        mesh=mesh,
        scratch_types=[
            pltpu.VMEM((NPAD // 32,), jnp.int32),
            pltpu.VMEM((GCH, D), jnp.float32),
            pltpu.VMEM((GCH, D), jnp.float32),
            pltpu.VMEM((GCH, D), jnp.float32),
            pltpu.VMEM((GCH, D), jnp.float32),
            pltpu.SemaphoreType.DMA,
            pltpu.SemaphoreType.DMA,
            pltpu.SemaphoreType.DMA,
            pltpu.SemaphoreType.DMA,
            pltpu.SemaphoreType.DMA,
            pltpu.SemaphoreType.DMA,
            pltpu.SemaphoreType.DMA,
            pltpu.SemaphoreType.DMA,
        ],
    )
    return f(src_idx, hf)


NSLOT = 3


def _k3_body(be_ref, runof_ref, runexp_ref, nruns_ref, x_ref,
             w1_hbm, w2_hbm, w3_hbm, o_ref, w1b, w2b, w3b, s1, s2, s3):
    b = pl.program_id(0)
    nruns = nruns_ref[0]

    def fetch(e, slot):
        pltpu.async_copy(w1_hbm.at[e], w1b.at[slot], s1.at[slot])
        pltpu.async_copy(w2_hbm.at[e], w2b.at[slot], s2.at[slot])
        pltpu.async_copy(w3_hbm.at[e], w3b.at[slot], s3.at[slot])

    def wait(slot):
        pltpu.make_async_copy(w1_hbm.at[0], w1b.at[slot], s1.at[slot]).wait()
        pltpu.make_async_copy(w2_hbm.at[0], w2b.at[slot], s2.at[slot]).wait()
        pltpu.make_async_copy(w3_hbm.at[0], w3b.at[slot], s3.at[slot]).wait()

    rc = runof_ref[b]

    @pl.when(b == 0)
    def _init():
        for r in range(NSLOT):
            @pl.when(r < nruns)
            def _():
                fetch(runexp_ref[r], r)

        wait(0)

    changed = jnp.logical_and(b > 0, rc != runof_ref[jnp.maximum(b - 1, 0)])

    @pl.when(changed)
    def _advance():
        pr = rc + NSLOT - 1

        @pl.when(pr < nruns)
        def _():
            fetch(runexp_ref[pr], lax.rem(pr, NSLOT))

        wait(lax.rem(rc, NSLOT))

    slot = lax.rem(rc, NSLOT)
    x = x_ref[...]
    z1 = _dot_nt(x, w1b[slot])
    z3 = _dot_nt(x, w3b[slot])
    act = z1 * jax.nn.sigmoid(z1) * z3
    o_ref[...] = _dot_nt(act, w2b[slot])


def _k3(block_expert, runof, runexp, nruns, xs, W1, W2, W3):
    grid_spec = pltpu.PrefetchScalarGridSpec(
        num_scalar_prefetch=4,
        grid=(NBLK,),
        in_specs=[
            pl.BlockSpec((BLK, D), lambda b, be, ro, re, nr: (b, 0)),
            pl.BlockSpec(memory_space=pl.ANY),
            pl.BlockSpec(memory_space=pl.ANY),
            pl.BlockSpec(memory_space=pl.ANY),
        ],
        out_specs=pl.BlockSpec((BLK, D), lambda b, be, ro, re, nr: (b, 0)),
        scratch_shapes=[
            pltpu.VMEM((NSLOT, D, D), jnp.float32),
            pltpu.VMEM((NSLOT, D, D), jnp.float32),
            pltpu.VMEM((NSLOT, D, D), jnp.float32),
            pltpu.SemaphoreType.DMA((NSLOT,)),
            pltpu.SemaphoreType.DMA((NSLOT,)),
            pltpu.SemaphoreType.DMA((NSLOT,)),
        ],
    )
    return pl.pallas_call(
        _k3_body,
        grid_spec=grid_spec,
        out_shape=jax.ShapeDtypeStruct((NPAD, D), jnp.float32),
    )(block_expert, runof, runexp, nruns, xs, W1, W2, W3)


CCH = 32


def _k4_body(outs_hbm, sh_hbm, slot_hbm, w_hbm, y_hbm,
             idx0_v, idx1_v, w0_v, w1_v, r0_v, r1_v, shv_v, y_v,
             sem, sem2, sem3, ysem):
    c = lax.axis_index("c")
    s = lax.axis_index("s")
    wid = s * 2 + c
    per = T // 32
    iota = lax.iota(jnp.int32, L)
    ycp = None
    for jc in range(per // CCH):
        t0 = wid * per + jc * CCH
        pltpu.sync_copy(slot_hbm.at[pl.ds(t0, CCH)], idx0_v)
        pltpu.sync_copy(slot_hbm.at[pl.ds(T + t0, CCH)], idx1_v)
        pltpu.sync_copy(w_hbm.at[pl.ds(t0, CCH)], w0_v)
        pltpu.sync_copy(w_hbm.at[pl.ds(T + t0, CCH)], w1_v)
        cp0 = pltpu.async_copy(outs_hbm.at[idx0_v], r0_v, sem)
        cp1 = pltpu.async_copy(outs_hbm.at[idx1_v], r1_v, sem2)
        cps = pltpu.async_copy(sh_hbm.at[pl.ds(t0, CCH), :], shv_v, sem3)
        cp0.wait()
        cp1.wait()
        cps.wait()
        if ycp is not None:
            ycp.wait()

        def tok(i, carry):
            g = i >> 4
            lane = jnp.bitwise_and(i, L - 1)
            w0g = w0_v[pl.ds(g * L, L)]
            w1g = w1_v[pl.ds(g * L, L)]
            w0s = jnp.full((L,), jnp.sum(jnp.where(iota == lane, w0g, 0.0)), jnp.float32)
            w1s = jnp.full((L,), jnp.sum(jnp.where(iota == lane, w1g, 0.0)), jnp.float32)

            def col(jj, carry2):
                sl = pl.ds(jj * L, L)
                y_v[i, sl] = (r0_v[i, sl] * w0s + r1_v[i, sl] * w1s
                              + shv_v[i, sl])
                return carry2

            lax.fori_loop(0, D // L, col, 0)
            return carry

        lax.fori_loop(0, CCH, tok, 0)
        ycp = pltpu.async_copy(y_v, y_hbm.at[pl.ds(t0, CCH), :], ysem)
    ycp.wait()


def _k4(outs, shared_y, slot_flat, w_flat):
    mesh = plsc.VectorSubcoreMesh(core_axis_name="c", subcore_axis_name="s")
    f = pl.kernel(
        _k4_body,
        compiler_params=pltpu.CompilerParams(needs_layout_passes=False),
        out_type=jax.ShapeDtypeStruct((T, D), jnp.float32),
        mesh=mesh,
        scratch_types=[
            pltpu.VMEM((CCH,), jnp.int32),
            pltpu.VMEM((CCH,), jnp.int32),
            pltpu.VMEM((CCH,), jnp.float32),
            pltpu.VMEM((CCH,), jnp.float32),
            pltpu.VMEM((CCH, D), jnp.float32),
            pltpu.VMEM((CCH, D), jnp.float32),
            pltpu.VMEM((CCH, D), jnp.float32),
            pltpu.VMEM((CCH, D), jnp.float32),
            pltpu.SemaphoreType.DMA,
            pltpu.SemaphoreType.DMA,
            pltpu.SemaphoreType.DMA,
            pltpu.SemaphoreType.DMA,
        ],
    )
    return f(outs, shared_y, slot_flat, w_flat)


def kernel(h, Wg, W1, W2, W3, W1s, W2s, W3s):
    b, s, d = h.shape
    hf = h.reshape(T, D)
    logits = _k1a(hf, Wg)
    shared_y = _k1b(hf, W1s, W2s, W3s)
    src_idx, slot_flat, w_flat, block_expert, runof, runexp, nruns = _k2(
        logits.reshape(-1))
    xs = _k2b(src_idx, hf)
    outs = _k3(block_expert, runof, runexp, nruns, xs, W1, W2, W3)
    y = _k4(outs, shared_y, slot_flat, w_flat)
    return y.reshape(b, s, d)


# speedup vs baseline: 1.0239x; 1.0219x over previous
import functools

import jax
import jax.numpy as jnp
from jax import lax
from jax.experimental import pallas as pl
from jax.experimental.pallas import tpu as pltpu
from jax.experimental.pallas import tpu_sc as plsc

T = 2048
D = 768
E = 16
NK = 2
P = 2 * T
BLK = 128
NPAD = P + E * BLK - 16
NPAD = 6144
NBLK = NPAD // BLK
L = 16


def _dot_nt(a, b):
    return lax.dot_general(a, b, (((1,), (1,)), ((), ())),
                           preferred_element_type=jnp.float32)


def _k1a_body(h_ref, wg_ref, lg_ref):
    lg_ref[...] = _dot_nt(h_ref[...], wg_ref[...])


def _k1a(hf, Wg, *, interpret=False):
    return pl.pallas_call(
        _k1a_body,
        grid=(1,),
        in_specs=[
            pl.BlockSpec((T, D), lambda t: (0, 0)),
            pl.BlockSpec((E, D), lambda t: (0, 0)),
        ],
        out_specs=pl.BlockSpec((T, E), lambda t: (0, 0)),
        out_shape=jax.ShapeDtypeStruct((T, E), jnp.float32),
        interpret=interpret,
    )(hf, Wg)


def _k1b_body(h_ref, w1s_ref, w2s_ref, w3s_ref, sh_ref):
    x = h_ref[...]
    z1 = _dot_nt(x, w1s_ref[...])
    z3 = _dot_nt(x, w3s_ref[...])
    act = z1 * jax.nn.sigmoid(z1) * z3
    sh_ref[...] = _dot_nt(act, w2s_ref[...])


def _k1b(hf, W1s, W2s, W3s, *, tt=256, interpret=False):
    grid = (T // tt,)
    return pl.pallas_call(
        _k1b_body,
        grid=grid,
        in_specs=[
            pl.BlockSpec((tt, D), lambda t: (t, 0)),
            pl.BlockSpec((D, D), lambda t: (0, 0)),
            pl.BlockSpec((D, D), lambda t: (0, 0)),
            pl.BlockSpec((D, D), lambda t: (0, 0)),
        ],
        out_specs=pl.BlockSpec((tt, D), lambda t: (t, 0)),
        out_shape=jax.ShapeDtypeStruct((T, D), jnp.float32),
        interpret=interpret,
    )(hf, W1s, W2s, W3s)


def _splat(v, lane):
    iota = lax.iota(jnp.int32, L)
    if v.dtype == jnp.int32:
        s = jnp.sum(jnp.where(iota == lane, v, 0))
    else:
        s = jnp.sum(jnp.where(iota == lane, v, 0.0))
    return jnp.full((L,), s, dtype=v.dtype)


def _k2_body(lg_hbm, src_hbm, slot_hbm, w_hbm, bexp_hbm, runof_hbm,
             runexp_hbm, nruns_hbm,
             lg_v, e0_v, e1_v, w0_v, w1_v, allids_v, cnts_v, c16_v,
             locsrc_v, locslot_v, acc_v, tmp_v, be_v, nx_v,
             nv_v, nv2_v, nr_v,
             ids_sp, cnt_sp, stsrc_sp, stslot_sp):
    c = lax.axis_index("c")
    s = lax.axis_index("s")
    on0 = c == 0
    iota = lax.iota(jnp.int32, L)

    @pl.when(on0)
    def _phase_a():
        pltpu.sync_copy(lg_hbm.at[pl.ds(s * (128 * E), 128 * E)], lg_v)

        def chunk(j, carry):
            base = j * L
            m1 = jnp.full((L,), -1e30, jnp.float32)
            m2 = jnp.full((L,), -1e30, jnp.float32)
            a1 = jnp.zeros((L,), jnp.int32)
            a2 = jnp.zeros((L,), jnp.int32)
            for e in range(E):
                ce = plsc.load_gather(lg_v, [(base + iota) * E + e])
                gt = ce > m1
                g2 = jnp.logical_and(jnp.logical_not(gt), ce > m2)
                m2n = jnp.where(gt, m1, jnp.where(g2, ce, m2))
                a2 = jnp.where(gt, a1, jnp.where(g2, e, a2))
                m2 = m2n
                a1 = jnp.where(gt, e, a1)
                m1 = jnp.where(gt, ce, m1)
            w0 = 1.0 / (1.0 + jnp.exp(m2 - m1))
            e0_v[pl.ds(base, L)] = a1
            e1_v[pl.ds(base, L)] = a2
            w0_v[pl.ds(base, L)] = w0
            w1_v[pl.ds(base, L)] = 1.0 - w0
            return carry

        lax.fori_loop(0, 128 // L, chunk, 0)
        pltpu.sync_copy(w0_v, w_hbm.at[pl.ds(s * 128, 128)])
        pltpu.sync_copy(w1_v, w_hbm.at[pl.ds(T + s * 128, 128)])
        pltpu.sync_copy(e0_v, ids_sp.at[pl.ds(s * 128, 128)])
        pltpu.sync_copy(e1_v, ids_sp.at[pl.ds(T + s * 128, 128)])

    plsc.subcore_barrier()

    @pl.when(on0)
    def _phase_b():
        pltpu.sync_copy(ids_sp, allids_v)

        def cb(i, cnt):
            v = allids_v[pl.ds(i * L, L)]
            return cnt + (v == s).astype(jnp.int32)

        cnt = lax.fori_loop(0, P // L, cb, jnp.zeros((L,), jnp.int32))
        tot = jnp.sum(cnt)
        c16_v[...] = jnp.full((L,), tot, jnp.int32)
        pltpu.sync_copy(c16_v, cnt_sp.at[s])

    plsc.subcore_barrier()

    @pl.when(on0)
    def _phase_c():
        pltpu.sync_copy(cnt_sp, cnts_v)
        counts = plsc.load_gather(cnts_v, [iota, iota])
        padded = ((counts + (BLK - 1)) >> 7) << 7
        cs = plsc.cumsum(padded)
        start = cs - padded

        def z1(i, carry):
            locsrc_v[pl.ds(i * L, L)] = jnp.zeros((L,), jnp.int32)
            return carry

        def z2(i, carry):
            locslot_v[pl.ds(i * L, L)] = jnp.zeros((L,), jnp.int32)
            return carry

        lax.fori_loop(0, NPAD // L, z1, 0)
        lax.fori_loop(0, P // L, z2, 0)

        start_s = _splat(start, s)

        def ce(i, rank):
            v = allids_v[pl.ds(i * L, L)]
            m = v == s
            mi = m.astype(jnp.int32)
            pcs = plsc.cumsum(mi)
            slots = rank + pcs - mi
            pairpos = i * L + iota
            tok = jnp.bitwise_and(pairpos, T - 1)
            plsc.store_scatter(locslot_v, [pairpos], slots, mask=m)
            plsc.store_scatter(locsrc_v, [slots], tok + 1, mask=m)
            return rank + _splat(pcs, L - 1)

        lax.fori_loop(0, P // L, ce, start_s)

        @pl.when(s == 0)
        def _bexp():
            startblk = start >> 7
            ne = (padded > 0).astype(jnp.int32)
            rank = plsc.cumsum(ne) - ne
            nv_v[...] = jnp.zeros((L,), jnp.int32)
            plsc.store_scatter(nv_v, [rank], iota, mask=padded > 0)
            nv2_v[...] = rank
            nr_v[...] = jnp.full((L,), jnp.sum(ne), jnp.int32)
            for cc in range(NBLK // L):
                bvec = iota + cc * L
                acc = jnp.zeros((L,), jnp.int32)
                for e in range(E):
                    sb = _splat(startblk, e)
                    pe = _splat(padded, e)
                    cond = jnp.logical_and(sb <= bvec, pe > 0)
                    acc = jnp.where(cond, e, acc)
                be_v[pl.ds(cc * L, L)] = acc
                nx_v[pl.ds(cc * L, L)] = plsc.load_gather(nv2_v, [acc])
            pltpu.sync_copy(be_v, bexp_hbm)
            pltpu.sync_copy(nx_v, runof_hbm)
            pltpu.sync_copy(nv_v, runexp_hbm)
            pltpu.sync_copy(nr_v, nruns_hbm)

        pltpu.sync_copy(locsrc_v, stsrc_sp.at[s])
        pltpu.sync_copy(locslot_v, stslot_sp.at[s])

    plsc.subcore_barrier()

    SRCW = NPAD // L
    SLTW = P // L

    @pl.when(on0)
    def _phase_d():
        pltpu.sync_copy(stsrc_sp.at[0, pl.ds(s * SRCW, SRCW)], acc_v.at[pl.ds(0, SRCW)])
        for r in range(1, L):
            pltpu.sync_copy(stsrc_sp.at[r, pl.ds(s * SRCW, SRCW)], tmp_v.at[pl.ds(0, SRCW)])
            for j in range(SRCW // L):
                acc_v[pl.ds(j * L, L)] = acc_v[pl.ds(j * L, L)] + tmp_v[pl.ds(j * L, L)]
        for j in range(SRCW // L):
            a = acc_v[pl.ds(j * L, L)]
            ramp = jnp.bitwise_and(s * SRCW + j * L + iota, T - 1)
            acc_v[pl.ds(j * L, L)] = jnp.where(a > 0, a - 1, ramp)
        pltpu.sync_copy(acc_v.at[pl.ds(0, SRCW)], src_hbm.at[pl.ds(s * SRCW, SRCW)])

        pltpu.sync_copy(stslot_sp.at[0, pl.ds(s * SLTW, SLTW)], acc_v.at[pl.ds(0, SLTW)])
        for r in range(1, L):
            pltpu.sync_copy(stslot_sp.at[r, pl.ds(s * SLTW, SLTW)], tmp_v.at[pl.ds(0, SLTW)])
            for j in range(SLTW // L):
                acc_v[pl.ds(j * L, L)] = acc_v[pl.ds(j * L, L)] + tmp_v[pl.ds(j * L, L)]
        pltpu.sync_copy(acc_v.at[pl.ds(0, SLTW)], slot_hbm.at[pl.ds(s * SLTW, SLTW)])


def _k2(logits_flat):
    mesh = plsc.VectorSubcoreMesh(core_axis_name="c", subcore_axis_name="s")
    f = pl.kernel(
        _k2_body,
        compiler_params=pltpu.CompilerParams(needs_layout_passes=False),
        out_type=[
            jax.ShapeDtypeStruct((NPAD,), jnp.int32),
            jax.ShapeDtypeStruct((P,), jnp.int32),
            jax.ShapeDtypeStruct((P,), jnp.float32),
            jax.ShapeDtypeStruct((NBLK,), jnp.int32),
            jax.ShapeDtypeStruct((NBLK,), jnp.int32),
            jax.ShapeDtypeStruct((L,), jnp.int32),
            jax.ShapeDtypeStruct((L,), jnp.int32),
        ],
        mesh=mesh,
        scratch_types=[
            pltpu.VMEM((128 * E,), jnp.float32),
            pltpu.VMEM((128,), jnp.int32),
            pltpu.VMEM((128,), jnp.int32),
            pltpu.VMEM((128,), jnp.float32),
            pltpu.VMEM((128,), jnp.float32),
            pltpu.VMEM((P,), jnp.int32),
            pltpu.VMEM((L, L), jnp.int32),
            pltpu.VMEM((L,), jnp.int32),
            pltpu.VMEM((NPAD,), jnp.int32),
            pltpu.VMEM((P,), jnp.int32),
            pltpu.VMEM((NPAD // L,), jnp.int32),
            pltpu.VMEM((NPAD // L,), jnp.int32),
            pltpu.VMEM((NBLK,), jnp.int32),
            pltpu.VMEM((NBLK,), jnp.int32),
            pltpu.VMEM((L,), jnp.int32),
            pltpu.VMEM((L,), jnp.int32),
            pltpu.VMEM((L,), jnp.int32),
            pltpu.VMEM_SHARED((P,), jnp.int32),
            pltpu.VMEM_SHARED((L, L), jnp.int32),
            pltpu.VMEM_SHARED((L, NPAD), jnp.int32),
            pltpu.VMEM_SHARED((L, P), jnp.int32),
        ],
    )
    return f(logits_flat)


GCH = 32
GNCH = 6
GNBUF = 4


def _k2b_body(src_hbm, hf_hbm, xs_hbm, idx_v, r0, r1, r2, r3,
              g0, g1, g2, g3, w0, w1, w2, w3):
    c = lax.axis_index("c")
    s = lax.axis_index("s")
    wid = s * 2 + c
    per = NPAD // 32
    base = wid * per
    bufs = (r0, r1, r2, r3)
    gsems = (g0, g1, g2, g3)
    wsems = (w0, w1, w2, w3)
    pltpu.sync_copy(src_hbm.at[pl.ds(base, per)], idx_v)
    gcp = [None] * GNCH
    wcp = [None] * GNCH
    for j in range(GNCH):
        b = j % GNBUF
        if j >= GNBUF:
            wcp[j - GNBUF].wait()
        gcp[j] = pltpu.async_copy(
            hf_hbm.at[idx_v.at[pl.ds(j * GCH, GCH)]], bufs[b], gsems[b])
        if j >= 1:
            bp = (j - 1) % GNBUF
            gcp[j - 1].wait()
            wcp[j - 1] = pltpu.async_copy(
                bufs[bp], xs_hbm.at[pl.ds(base + (j - 1) * GCH, GCH), :],
                wsems[bp])
    gcp[GNCH - 1].wait()
    wcp[GNCH - 1] = pltpu.async_copy(
        bufs[(GNCH - 1) % GNBUF],
        xs_hbm.at[pl.ds(base + (GNCH - 1) * GCH, GCH), :],
        wsems[(GNCH - 1) % GNBUF])
    for j in range(GNCH - GNBUF, GNCH):
        if j >= 0 and wcp[j] is not None:
            wcp[j].wait()


def _k2b(src_idx, hf):
    mesh = plsc.VectorSubcoreMesh(core_axis_name="c", subcore_axis_name="s")
    f = pl.kernel(
        _k2b_body,
        compiler_params=pltpu.CompilerParams(needs_layout_passes=False),
        out_type=jax.ShapeDtypeStruct((NPAD, D), jnp.float32),
        mesh=mesh,
        scratch_types=[
            pltpu.VMEM((NPAD // 32,), jnp.int32),
            pltpu.VMEM((GCH, D), jnp.float32),
            pltpu.VMEM((GCH, D), jnp.float32),
            pltpu.VMEM((GCH, D), jnp.float32),
            pltpu.VMEM((GCH, D), jnp.float32),
            pltpu.SemaphoreType.DMA,
            pltpu.SemaphoreType.DMA,
            pltpu.SemaphoreType.DMA,
            pltpu.SemaphoreType.DMA,
            pltpu.SemaphoreType.DMA,
            pltpu.SemaphoreType.DMA,
            pltpu.SemaphoreType.DMA,
            pltpu.SemaphoreType.DMA,
        ],
    )
    return f(src_idx, hf)


NSLOT = 3


def _k3_body(be_ref, runof_ref, runexp_ref, nruns_ref, x_ref,
             w1_hbm, w2_hbm, w3_hbm, o_ref, w1b, w2b, w3b, s1, s2, s3):
    b = pl.program_id(0)
    nruns = nruns_ref[0]

    def fetch(e, slot):
        pltpu.async_copy(w1_hbm.at[e], w1b.at[slot], s1.at[slot])
        pltpu.async_copy(w2_hbm.at[e], w2b.at[slot], s2.at[slot])
        pltpu.async_copy(w3_hbm.at[e], w3b.at[slot], s3.at[slot])

    def wait(slot):
        pltpu.make_async_copy(w1_hbm.at[0], w1b.at[slot], s1.at[slot]).wait()
        pltpu.make_async_copy(w2_hbm.at[0], w2b.at[slot], s2.at[slot]).wait()
        pltpu.make_async_copy(w3_hbm.at[0], w3b.at[slot], s3.at[slot]).wait()

    rc = runof_ref[b]

    @pl.when(b == 0)
    def _init():
        for r in range(NSLOT):
            @pl.when(r < nruns)
            def _():
                fetch(runexp_ref[r], r)

        wait(0)

    changed = jnp.logical_and(b > 0, rc != runof_ref[jnp.maximum(b - 1, 0)])

    @pl.when(changed)
    def _advance():
        pr = rc + NSLOT - 1

        @pl.when(pr < nruns)
        def _():
            fetch(runexp_ref[pr], lax.rem(pr, NSLOT))

        wait(lax.rem(rc, NSLOT))

    slot = lax.rem(rc, NSLOT)
    x = x_ref[...]
    z1 = _dot_nt(x, w1b[slot])
    z3 = _dot_nt(x, w3b[slot])
    act = z1 * jax.nn.sigmoid(z1) * z3
    o_ref[...] = _dot_nt(act, w2b[slot])


def _k3(block_expert, runof, runexp, nruns, xs, W1, W2, W3):
    grid_spec = pltpu.PrefetchScalarGridSpec(
        num_scalar_prefetch=4,
        grid=(NBLK,),
        in_specs=[
            pl.BlockSpec((BLK, D), lambda b, be, ro, re, nr: (b, 0)),
            pl.BlockSpec(memory_space=pl.ANY),
            pl.BlockSpec(memory_space=pl.ANY),
            pl.BlockSpec(memory_space=pl.ANY),
        ],
        out_specs=pl.BlockSpec((BLK, D), lambda b, be, ro, re, nr: (b, 0)),
        scratch_shapes=[
            pltpu.VMEM((NSLOT, D, D), jnp.float32),
            pltpu.VMEM((NSLOT, D, D), jnp.float32),
            pltpu.VMEM((NSLOT, D, D), jnp.float32),
            pltpu.SemaphoreType.DMA((NSLOT,)),
            pltpu.SemaphoreType.DMA((NSLOT,)),
            pltpu.SemaphoreType.DMA((NSLOT,)),
        ],
    )
    return pl.pallas_call(
        _k3_body,
        grid_spec=grid_spec,
        out_shape=jax.ShapeDtypeStruct((NPAD, D), jnp.float32),
    )(block_expert, runof, runexp, nruns, xs, W1, W2, W3)


CCH = 32


def _k4_body(outs_hbm, sh_hbm, slot_hbm, w_hbm, y_hbm,
             idx0_v, idx1_v, w0_v, w1_v, r0_v, r1_v, shv_v, y_v,
             sem, sem2, sem3, ysem):
    c = lax.axis_index("c")
    s = lax.axis_index("s")
    wid = s * 2 + c
    per = T // 32
    iota = lax.iota(jnp.int32, L)
    ycp = None
    for jc in range(per // CCH):
        t0 = wid * per + jc * CCH
        pltpu.sync_copy(slot_hbm.at[pl.ds(t0, CCH)], idx0_v)
        pltpu.sync_copy(slot_hbm.at[pl.ds(T + t0, CCH)], idx1_v)
        pltpu.sync_copy(w_hbm.at[pl.ds(t0, CCH)], w0_v)
        pltpu.sync_copy(w_hbm.at[pl.ds(T + t0, CCH)], w1_v)
        cp0 = pltpu.async_copy(outs_hbm.at[idx0_v], r0_v, sem)
        cp1 = pltpu.async_copy(outs_hbm.at[idx1_v], r1_v, sem2)
        cps = pltpu.async_copy(sh_hbm.at[pl.ds(t0, CCH), :], shv_v, sem3)
        cp0.wait()
        cp1.wait()
        cps.wait()
        if ycp is not None:
            ycp.wait()

        def tok(i, carry):
            g = i >> 4
            lane = jnp.bitwise_and(i, L - 1)
            w0g = w0_v[pl.ds(g * L, L)]
            w1g = w1_v[pl.ds(g * L, L)]
            w0s = jnp.full((L,), jnp.sum(jnp.where(iota == lane, w0g, 0.0)), jnp.float32)
            w1s = jnp.full((L,), jnp.sum(jnp.where(iota == lane, w1g, 0.0)), jnp.float32)

            def col(jj, carry2):
                sl = pl.ds(jj * L, L)
                y_v[i, sl] = (r0_v[i, sl] * w0s + r1_v[i, sl] * w1s
                              + shv_v[i, sl])
                return carry2

            lax.fori_loop(0, D // L, col, 0)
            return carry

        lax.fori_loop(0, CCH, tok, 0)
        ycp = pltpu.async_copy(y_v, y_hbm.at[pl.ds(t0, CCH), :], ysem)
    ycp.wait()


def _k4(outs, shared_y, slot_flat, w_flat):
    mesh = plsc.VectorSubcoreMesh(core_axis_name="c", subcore_axis_name="s")
    f = pl.kernel(
        _k4_body,
        compiler_params=pltpu.CompilerParams(needs_layout_passes=False),
        out_type=jax.ShapeDtypeStruct((T, D), jnp.float32),
        mesh=mesh,
        scratch_types=[
            pltpu.VMEM((CCH,), jnp.int32),
            pltpu.VMEM((CCH,), jnp.int32),
            pltpu.VMEM((CCH,), jnp.float32),
            pltpu.VMEM((CCH,), jnp.float32),
            pltpu.VMEM((CCH, D), jnp.float32),
            pltpu.VMEM((CCH, D), jnp.float32),
            pltpu.VMEM((CCH, D), jnp.float32),
            pltpu.VMEM((CCH, D), jnp.float32),
            pltpu.SemaphoreType.DMA,
            pltpu.SemaphoreType.DMA,
            pltpu.SemaphoreType.DMA,
            pltpu.SemaphoreType.DMA,
        ],
    )
    return f(outs, shared_y, slot_flat, w_flat)


def kernel(h, Wg, W1, W2, W3, W1s, W2s, W3s):
    b, s, d = h.shape
    hf = h.reshape(T, D)
    logits = _k1a(hf, Wg)
    shared_y = _k1b(hf, W1s, W2s, W3s)
    src_idx, slot_flat, w_flat, block_expert, runof, runexp, nruns = _k2(
        logits.reshape(-1))
    xs = _k2b(src_idx, hf)
    outs = _k3(block_expert, runof, runexp, nruns, xs, W1, W2, W3)
    y = _k4(outs, shared_y, slot_flat, w_flat)
    return y.reshape(b, s, d)


# speedup vs baseline: 1.0545x; 1.0298x over previous
import functools

import jax
import jax.numpy as jnp
from jax import lax
from jax.experimental import pallas as pl
from jax.experimental.pallas import tpu as pltpu
from jax.experimental.pallas import tpu_sc as plsc

T = 2048
D = 768
E = 16
NK = 2
P = 2 * T
BLK = 128
NPAD = P + E * BLK - 16
NPAD = 6144
NBLK = NPAD // BLK
L = 16


def _dot_nt(a, b):
    return lax.dot_general(a, b, (((1,), (1,)), ((), ())),
                           preferred_element_type=jnp.float32)


def _k1a_body(h_ref, wg_ref, lg_ref):
    lg_ref[...] = _dot_nt(h_ref[...], wg_ref[...])


def _k1a(hf, Wg, *, interpret=False):
    return pl.pallas_call(
        _k1a_body,
        grid=(1,),
        in_specs=[
            pl.BlockSpec((T, D), lambda t: (0, 0)),
            pl.BlockSpec((E, D), lambda t: (0, 0)),
        ],
        out_specs=pl.BlockSpec((T, E), lambda t: (0, 0)),
        out_shape=jax.ShapeDtypeStruct((T, E), jnp.float32),
        interpret=interpret,
    )(hf, Wg)


def _k1b_body(h_ref, w1s_ref, w2s_ref, w3s_ref, sh_ref):
    x = h_ref[...]
    z1 = _dot_nt(x, w1s_ref[...])
    z3 = _dot_nt(x, w3s_ref[...])
    act = z1 * jax.nn.sigmoid(z1) * z3
    sh_ref[...] = _dot_nt(act, w2s_ref[...])


def _k1b(hf, W1s, W2s, W3s, *, tt=256, interpret=False):
    grid = (T // tt,)
    return pl.pallas_call(
        _k1b_body,
        grid=grid,
        in_specs=[
            pl.BlockSpec((tt, D), lambda t: (t, 0)),
            pl.BlockSpec((D, D), lambda t: (0, 0)),
            pl.BlockSpec((D, D), lambda t: (0, 0)),
            pl.BlockSpec((D, D), lambda t: (0, 0)),
        ],
        out_specs=pl.BlockSpec((tt, D), lambda t: (t, 0)),
        out_shape=jax.ShapeDtypeStruct((T, D), jnp.float32),
        interpret=interpret,
    )(hf, W1s, W2s, W3s)


def _splat(v, lane):
    iota = lax.iota(jnp.int32, L)
    if v.dtype == jnp.int32:
        s = jnp.sum(jnp.where(iota == lane, v, 0))
    else:
        s = jnp.sum(jnp.where(iota == lane, v, 0.0))
    return jnp.full((L,), s, dtype=v.dtype)


def _k2_body(lg_hbm, src_hbm, slot_hbm, w_hbm, bexp_hbm, runof_hbm,
             runexp_hbm, nruns_hbm,
             lg_v, e0_v, e1_v, w0_v, w1_v, allids_v, cnts_v, c16_v,
             locsrc_v, locslot_v, stga_v, stgb_v, be_v, nx_v,
             nv_v, nv2_v, nr_v,
             ids_sp, cnt_sp, stsrc_sp, stslot_sp):
    c = lax.axis_index("c")
    s = lax.axis_index("s")
    on0 = c == 0
    iota = lax.iota(jnp.int32, L)

    @pl.when(on0)
    def _phase_a():
        pltpu.sync_copy(lg_hbm.at[pl.ds(s * (128 * E), 128 * E)], lg_v)

        def chunk(j, carry):
            base = j * L
            m1 = jnp.full((L,), -1e30, jnp.float32)
            m2 = jnp.full((L,), -1e30, jnp.float32)
            a1 = jnp.zeros((L,), jnp.int32)
            a2 = jnp.zeros((L,), jnp.int32)
            for e in range(E):
                ce = plsc.load_gather(lg_v, [(base + iota) * E + e])
                gt = ce > m1
                g2 = jnp.logical_and(jnp.logical_not(gt), ce > m2)
                m2n = jnp.where(gt, m1, jnp.where(g2, ce, m2))
                a2 = jnp.where(gt, a1, jnp.where(g2, e, a2))
                m2 = m2n
                a1 = jnp.where(gt, e, a1)
                m1 = jnp.where(gt, ce, m1)
            w0 = 1.0 / (1.0 + jnp.exp(m2 - m1))
            e0_v[pl.ds(base, L)] = a1
            e1_v[pl.ds(base, L)] = a2
            w0_v[pl.ds(base, L)] = w0
            w1_v[pl.ds(base, L)] = 1.0 - w0
            return carry

        lax.fori_loop(0, 128 // L, chunk, 0)
        pltpu.sync_copy(w0_v, w_hbm.at[pl.ds(s * 128, 128)])
        pltpu.sync_copy(w1_v, w_hbm.at[pl.ds(T + s * 128, 128)])
        pltpu.sync_copy(e0_v, ids_sp.at[pl.ds(s * 128, 128)])
        pltpu.sync_copy(e1_v, ids_sp.at[pl.ds(T + s * 128, 128)])

    plsc.subcore_barrier()

    @pl.when(on0)
    def _phase_b():
        pltpu.sync_copy(ids_sp, allids_v)

        def cb(i, cnt):
            v = allids_v[pl.ds(i * L, L)]
            return cnt + (v == s).astype(jnp.int32)

        cnt = lax.fori_loop(0, P // L, cb, jnp.zeros((L,), jnp.int32))
        tot = jnp.sum(cnt)
        c16_v[...] = jnp.full((L,), tot, jnp.int32)
        pltpu.sync_copy(c16_v, cnt_sp.at[s])

    plsc.subcore_barrier()

    @pl.when(on0)
    def _phase_c():
        pltpu.sync_copy(cnt_sp, cnts_v)
        counts = plsc.load_gather(cnts_v, [iota, iota])
        padded = ((counts + (BLK - 1)) >> 7) << 7
        cs = plsc.cumsum(padded)
        start = cs - padded

        def z1(i, carry):
            locsrc_v[pl.ds(i * L, L)] = jnp.zeros((L,), jnp.int32)
            return carry

        def z2(i, carry):
            locslot_v[pl.ds(i * L, L)] = jnp.zeros((L,), jnp.int32)
            return carry

        lax.fori_loop(0, NPAD // L, z1, 0)
        lax.fori_loop(0, P // L, z2, 0)

        start_s = _splat(start, s)

        def ce(i, rank):
            v = allids_v[pl.ds(i * L, L)]
            m = v == s
            mi = m.astype(jnp.int32)
            pcs = plsc.cumsum(mi)
            slots = rank + pcs - mi
            pairpos = i * L + iota
            tok = jnp.bitwise_and(pairpos, T - 1)
            plsc.store_scatter(locslot_v, [pairpos], slots, mask=m)
            plsc.store_scatter(locsrc_v, [slots], tok + 1, mask=m)
            return rank + _splat(pcs, L - 1)

        lax.fori_loop(0, P // L, ce, start_s)

        @pl.when(s == 0)
        def _bexp():
            startblk = start >> 7
            ne = (padded > 0).astype(jnp.int32)
            rank = plsc.cumsum(ne) - ne
            nv_v[...] = jnp.zeros((L,), jnp.int32)
            plsc.store_scatter(nv_v, [rank], iota, mask=padded > 0)
            nv2_v[...] = rank
            nr_v[...] = jnp.full((L,), jnp.sum(ne), jnp.int32)
            for cc in range(NBLK // L):
                bvec = iota + cc * L
                acc = jnp.zeros((L,), jnp.int32)
                for e in range(E):
                    sb = _splat(startblk, e)
                    pe = _splat(padded, e)
                    cond = jnp.logical_and(sb <= bvec, pe > 0)
                    acc = jnp.where(cond, e, acc)
                be_v[pl.ds(cc * L, L)] = acc
                nx_v[pl.ds(cc * L, L)] = plsc.load_gather(nv2_v, [acc])
            pltpu.sync_copy(be_v, bexp_hbm)
            pltpu.sync_copy(nx_v, runof_hbm)
            pltpu.sync_copy(nv_v, runexp_hbm)
            pltpu.sync_copy(nr_v, nruns_hbm)

        pltpu.sync_copy(locsrc_v, stsrc_sp.at[s])
        pltpu.sync_copy(locslot_v, stslot_sp.at[s])

    plsc.subcore_barrier()

    SRCW = NPAD // L
    SLTW = P // L

    @pl.when(on0)
    def _phase_d():
        pltpu.sync_copy(stsrc_sp.at[:, pl.ds(s * SRCW, SRCW)], stga_v)
        pltpu.sync_copy(stslot_sp.at[:, pl.ds(s * SLTW, SLTW)], stgb_v)

        def addj(j, carry):
            acc = stga_v[0, pl.ds(j * L, L)]
            for r in range(1, L):
                acc = acc + stga_v[r, pl.ds(j * L, L)]
            ramp = jnp.bitwise_and(s * SRCW + j * L + iota, T - 1)
            stga_v[0, pl.ds(j * L, L)] = jnp.where(acc > 0, acc - 1, ramp)
            return carry

        lax.fori_loop(0, SRCW // L, addj, 0)
        pltpu.sync_copy(stga_v.at[0], src_hbm.at[pl.ds(s * SRCW, SRCW)])

        def addj2(j, carry):
            acc = stgb_v[0, pl.ds(j * L, L)]
            for r in range(1, L):
                acc = acc + stgb_v[r, pl.ds(j * L, L)]
            stgb_v[0, pl.ds(j * L, L)] = acc
            return carry

        lax.fori_loop(0, SLTW // L, addj2, 0)
        pltpu.sync_copy(stgb_v.at[0], slot_hbm.at[pl.ds(s * SLTW, SLTW)])


def _k2(logits_flat):
    mesh = plsc.VectorSubcoreMesh(core_axis_name="c", subcore_axis_name="s")
    f = pl.kernel(
        _k2_body,
        compiler_params=pltpu.CompilerParams(needs_layout_passes=False),
        out_type=[
            jax.ShapeDtypeStruct((NPAD,), jnp.int32),
            jax.ShapeDtypeStruct((P,), jnp.int32),
            jax.ShapeDtypeStruct((P,), jnp.float32),
            jax.ShapeDtypeStruct((NBLK,), jnp.int32),
            jax.ShapeDtypeStruct((NBLK,), jnp.int32),
            jax.ShapeDtypeStruct((L,), jnp.int32),
            jax.ShapeDtypeStruct((L,), jnp.int32),
        ],
        mesh=mesh,
        scratch_types=[
            pltpu.VMEM((128 * E,), jnp.float32),
            pltpu.VMEM((128,), jnp.int32),
            pltpu.VMEM((128,), jnp.int32),
            pltpu.VMEM((128,), jnp.float32),
            pltpu.VMEM((128,), jnp.float32),
            pltpu.VMEM((P,), jnp.int32),
            pltpu.VMEM((L, L), jnp.int32),
            pltpu.VMEM((L,), jnp.int32),
            pltpu.VMEM((NPAD,), jnp.int32),
            pltpu.VMEM((P,), jnp.int32),
            pltpu.VMEM((L, NPAD // L), jnp.int32),
            pltpu.VMEM((L, P // L), jnp.int32),
            pltpu.VMEM((NBLK,), jnp.int32),
            pltpu.VMEM((NBLK,), jnp.int32),
            pltpu.VMEM((L,), jnp.int32),
            pltpu.VMEM((L,), jnp.int32),
            pltpu.VMEM((L,), jnp.int32),
            pltpu.VMEM_SHARED((P,), jnp.int32),
            pltpu.VMEM_SHARED((L, L), jnp.int32),
            pltpu.VMEM_SHARED((L, NPAD), jnp.int32),
            pltpu.VMEM_SHARED((L, P), jnp.int32),
        ],
    )
    return f(logits_flat)


GCH = 32
GNCH = 6
GNBUF = 4


def _k2b_body(src_hbm, hf_hbm, xs_hbm, idx_v, r0, r1, r2, r3,
              g0, g1, g2, g3, w0, w1, w2, w3):
    c = lax.axis_index("c")
    s = lax.axis_index("s")
    wid = s * 2 + c
    per = NPAD // 32
    base = wid * per
    bufs = (r0, r1, r2, r3)
    gsems = (g0, g1, g2, g3)
    wsems = (w0, w1, w2, w3)
    pltpu.sync_copy(src_hbm.at[pl.ds(base, per)], idx_v)
    gcp = [None] * GNCH
    wcp = [None] * GNCH
    for j in range(GNCH):
        b = j % GNBUF
        if j >= GNBUF:
            wcp[j - GNBUF].wait()
        gcp[j] = pltpu.async_copy(
            hf_hbm.at[idx_v.at[pl.ds(j * GCH, GCH)]], bufs[b], gsems[b])
        if j >= 1:
            bp = (j - 1) % GNBUF
            gcp[j - 1].wait()
            wcp[j - 1] = pltpu.async_copy(
                bufs[bp], xs_hbm.at[pl.ds(base + (j - 1) * GCH, GCH), :],
                wsems[bp])
    gcp[GNCH - 1].wait()
    wcp[GNCH - 1] = pltpu.async_copy(
        bufs[(GNCH - 1) % GNBUF],
        xs_hbm.at[pl.ds(base + (GNCH - 1) * GCH, GCH), :],
        wsems[(GNCH - 1) % GNBUF])
    for j in range(GNCH - GNBUF, GNCH):
        if j >= 0 and wcp[j] is not None:
            wcp[j].wait()


def _k2b(src_idx, hf):
    mesh = plsc.VectorSubcoreMesh(core_axis_name="c", subcore_axis_name="s")
    f = pl.kernel(
        _k2b_body,
        compiler_params=pltpu.CompilerParams(needs_layout_passes=False),
        out_type=jax.ShapeDtypeStruct((NPAD, D), jnp.float32),
        mesh=mesh,
        scratch_types=[
            pltpu.VMEM((NPAD // 32,), jnp.int32),
            pltpu.VMEM((GCH, D), jnp.float32),
            pltpu.VMEM((GCH, D), jnp.float32),
            pltpu.VMEM((GCH, D), jnp.float32),
            pltpu.VMEM((GCH, D), jnp.float32),
            pltpu.SemaphoreType.DMA,
            pltpu.SemaphoreType.DMA,
            pltpu.SemaphoreType.DMA,
            pltpu.SemaphoreType.DMA,
            pltpu.SemaphoreType.DMA,
            pltpu.SemaphoreType.DMA,
            pltpu.SemaphoreType.DMA,
            pltpu.SemaphoreType.DMA,
        ],
    )
    return f(src_idx, hf)


NSLOT = 3


def _k3_body(be_ref, runof_ref, runexp_ref, nruns_ref, x_ref,
             w1_hbm, w2_hbm, w3_hbm, o_ref, w1b, w2b, w3b, s1, s2, s3):
    b = pl.program_id(0)
    nruns = nruns_ref[0]

    def fetch(e, slot):
        pltpu.async_copy(w1_hbm.at[e], w1b.at[slot], s1.at[slot])
        pltpu.async_copy(w2_hbm.at[e], w2b.at[slot], s2.at[slot])
        pltpu.async_copy(w3_hbm.at[e], w3b.at[slot], s3.at[slot])

    def wait(slot):
        pltpu.make_async_copy(w1_hbm.at[0], w1b.at[slot], s1.at[slot]).wait()
        pltpu.make_async_copy(w2_hbm.at[0], w2b.at[slot], s2.at[slot]).wait()
        pltpu.make_async_copy(w3_hbm.at[0], w3b.at[slot], s3.at[slot]).wait()

    rc = runof_ref[b]

    @pl.when(b == 0)
    def _init():
        for r in range(NSLOT):
            @pl.when(r < nruns)
            def _():
                fetch(runexp_ref[r], r)

        wait(0)

    changed = jnp.logical_and(b > 0, rc != runof_ref[jnp.maximum(b - 1, 0)])

    @pl.when(changed)
    def _advance():
        pr = rc + NSLOT - 1

        @pl.when(pr < nruns)
        def _():
            fetch(runexp_ref[pr], lax.rem(pr, NSLOT))

        wait(lax.rem(rc, NSLOT))

    slot = lax.rem(rc, NSLOT)
    x = x_ref[...]
    z1 = _dot_nt(x, w1b[slot])
    z3 = _dot_nt(x, w3b[slot])
    act = z1 * jax.nn.sigmoid(z1) * z3
    o_ref[...] = _dot_nt(act, w2b[slot])


def _k3(block_expert, runof, runexp, nruns, xs, W1, W2, W3):
    grid_spec = pltpu.PrefetchScalarGridSpec(
        num_scalar_prefetch=4,
        grid=(NBLK,),
        in_specs=[
            pl.BlockSpec((BLK, D), lambda b, be, ro, re, nr: (b, 0)),
            pl.BlockSpec(memory_space=pl.ANY),
            pl.BlockSpec(memory_space=pl.ANY),
            pl.BlockSpec(memory_space=pl.ANY),
        ],
        out_specs=pl.BlockSpec((BLK, D), lambda b, be, ro, re, nr: (b, 0)),
        scratch_shapes=[
            pltpu.VMEM((NSLOT, D, D), jnp.float32),
            pltpu.VMEM((NSLOT, D, D), jnp.float32),
            pltpu.VMEM((NSLOT, D, D), jnp.float32),
            pltpu.SemaphoreType.DMA((NSLOT,)),
            pltpu.SemaphoreType.DMA((NSLOT,)),
            pltpu.SemaphoreType.DMA((NSLOT,)),
        ],
    )
    return pl.pallas_call(
        _k3_body,
        grid_spec=grid_spec,
        out_shape=jax.ShapeDtypeStruct((NPAD, D), jnp.float32),
    )(block_expert, runof, runexp, nruns, xs, W1, W2, W3)


CCH = 32


def _k4_body(outs_hbm, sh_hbm, slot_hbm, w_hbm, y_hbm,
             idx0_v, idx1_v, w0_v, w1_v, r0_v, r1_v, shv_v, y_v,
             sem, sem2, sem3, ysem):
    c = lax.axis_index("c")
    s = lax.axis_index("s")
    wid = s * 2 + c
    per = T // 32
    iota = lax.iota(jnp.int32, L)
    ycp = None
    for jc in range(per // CCH):
        t0 = wid * per + jc * CCH
        pltpu.sync_copy(slot_hbm.at[pl.ds(t0, CCH)], idx0_v)
        pltpu.sync_copy(slot_hbm.at[pl.ds(T + t0, CCH)], idx1_v)
        pltpu.sync_copy(w_hbm.at[pl.ds(t0, CCH)], w0_v)
        pltpu.sync_copy(w_hbm.at[pl.ds(T + t0, CCH)], w1_v)
        cp0 = pltpu.async_copy(outs_hbm.at[idx0_v], r0_v, sem)
        cp1 = pltpu.async_copy(outs_hbm.at[idx1_v], r1_v, sem2)
        cps = pltpu.async_copy(sh_hbm.at[pl.ds(t0, CCH), :], shv_v, sem3)
        cp0.wait()
        cp1.wait()
        cps.wait()
        if ycp is not None:
            ycp.wait()

        def tok(i, carry):
            g = i >> 4
            lane = jnp.bitwise_and(i, L - 1)
            w0g = w0_v[pl.ds(g * L, L)]
            w1g = w1_v[pl.ds(g * L, L)]
            w0s = jnp.full((L,), jnp.sum(jnp.where(iota == lane, w0g, 0.0)), jnp.float32)
            w1s = jnp.full((L,), jnp.sum(jnp.where(iota == lane, w1g, 0.0)), jnp.float32)

            def col(jj, carry2):
                sl = pl.ds(jj * L, L)
                y_v[i, sl] = (r0_v[i, sl] * w0s + r1_v[i, sl] * w1s
                              + shv_v[i, sl])
                return carry2

            lax.fori_loop(0, D // L, col, 0)
            return carry

        lax.fori_loop(0, CCH, tok, 0)
        ycp = pltpu.async_copy(y_v, y_hbm.at[pl.ds(t0, CCH), :], ysem)
    ycp.wait()


def _k4(outs, shared_y, slot_flat, w_flat):
    mesh = plsc.VectorSubcoreMesh(core_axis_name="c", subcore_axis_name="s")
    f = pl.kernel(
        _k4_body,
        compiler_params=pltpu.CompilerParams(needs_layout_passes=False),
        out_type=jax.ShapeDtypeStruct((T, D), jnp.float32),
        mesh=mesh,
        scratch_types=[
            pltpu.VMEM((CCH,), jnp.int32),
            pltpu.VMEM((CCH,), jnp.int32),
            pltpu.VMEM((CCH,), jnp.float32),
            pltpu.VMEM((CCH,), jnp.float32),
            pltpu.VMEM((CCH, D), jnp.float32),
            pltpu.VMEM((CCH, D), jnp.float32),
            pltpu.VMEM((CCH, D), jnp.float32),
            pltpu.VMEM((CCH, D), jnp.float32),
            pltpu.SemaphoreType.DMA,
            pltpu.SemaphoreType.DMA,
            pltpu.SemaphoreType.DMA,
            pltpu.SemaphoreType.DMA,
        ],
    )
    return f(outs, shared_y, slot_flat, w_flat)


def kernel(h, Wg, W1, W2, W3, W1s, W2s, W3s):
    b, s, d = h.shape
    hf = h.reshape(T, D)
    logits = _k1a(hf, Wg)
    shared_y = _k1b(hf, W1s, W2s, W3s)
    src_idx, slot_flat, w_flat, block_expert, runof, runexp, nruns = _k2(
        logits.reshape(-1))
    xs = _k2b(src_idx, hf)
    outs = _k3(block_expert, runof, runexp, nruns, xs, W1, W2, W3)
    y = _k4(outs, shared_y, slot_flat, w_flat)
    return y.reshape(b, s, d)


# speedup vs baseline: 1.1021x; 1.0452x over previous
import functools

import jax
import jax.numpy as jnp
from jax import lax
from jax.experimental import pallas as pl
from jax.experimental.pallas import tpu as pltpu
from jax.experimental.pallas import tpu_sc as plsc

T = 2048
D = 768
E = 16
NK = 2
P = 2 * T
BLK = 128
NPAD = P + E * BLK - 16
NPAD = 6144
NBLK = NPAD // BLK
L = 16


def _dot_nt(a, b):
    return lax.dot_general(a, b, (((1,), (1,)), ((), ())),
                           preferred_element_type=jnp.float32)


def _k1a_body(h_ref, wg_ref, lg_ref):
    lg_ref[...] = _dot_nt(h_ref[...], wg_ref[...])


def _k1a(hf, Wg, *, interpret=False):
    return pl.pallas_call(
        _k1a_body,
        grid=(1,),
        in_specs=[
            pl.BlockSpec((T, D), lambda t: (0, 0)),
            pl.BlockSpec((E, D), lambda t: (0, 0)),
        ],
        out_specs=pl.BlockSpec((T, E), lambda t: (0, 0)),
        out_shape=jax.ShapeDtypeStruct((T, E), jnp.float32),
        interpret=interpret,
    )(hf, Wg)


def _k1b_body(h_ref, w1s_ref, w2s_ref, w3s_ref, sh_ref):
    x = h_ref[...]
    z1 = _dot_nt(x, w1s_ref[...])
    z3 = _dot_nt(x, w3s_ref[...])
    act = z1 * jax.nn.sigmoid(z1) * z3
    sh_ref[...] = _dot_nt(act, w2s_ref[...])


def _k1b(hf, W1s, W2s, W3s, *, tt=256, interpret=False):
    grid = (T // tt,)
    return pl.pallas_call(
        _k1b_body,
        grid=grid,
        in_specs=[
            pl.BlockSpec((tt, D), lambda t: (t, 0)),
            pl.BlockSpec((D, D), lambda t: (0, 0)),
            pl.BlockSpec((D, D), lambda t: (0, 0)),
            pl.BlockSpec((D, D), lambda t: (0, 0)),
        ],
        out_specs=pl.BlockSpec((tt, D), lambda t: (t, 0)),
        out_shape=jax.ShapeDtypeStruct((T, D), jnp.float32),
        interpret=interpret,
    )(hf, W1s, W2s, W3s)


def _splat(v, lane):
    iota = lax.iota(jnp.int32, L)
    if v.dtype == jnp.int32:
        s = jnp.sum(jnp.where(iota == lane, v, 0))
    else:
        s = jnp.sum(jnp.where(iota == lane, v, 0.0))
    return jnp.full((L,), s, dtype=v.dtype)


def _k2_body(lg_hbm, src_hbm, slot_hbm, w_hbm, bexp_hbm, runof_hbm,
             runexp_hbm, nruns_hbm,
             lg_v, e0_v, e1_v, w0_v, w1_v, allids_v, cnts_v, c16_v,
             locsrc_v, locslot_v, stga_v, stgb_v, be_v, nx_v,
             nv_v, nv2_v, nr_v,
             ids_sp, cnt_sp, stsrc_sp, stslot_sp):
    c = lax.axis_index("c")
    s = lax.axis_index("s")
    on0 = c == 0
    iota = lax.iota(jnp.int32, L)

    @pl.when(on0)
    def _phase_a():
        pltpu.sync_copy(lg_hbm.at[pl.ds(s * (128 * E), 128 * E)], lg_v)

        def chunk(j, carry):
            base = j * L
            m1 = jnp.full((L,), -1e30, jnp.float32)
            m2 = jnp.full((L,), -1e30, jnp.float32)
            a1 = jnp.zeros((L,), jnp.int32)
            a2 = jnp.zeros((L,), jnp.int32)
            for e in range(E):
                ce = plsc.load_gather(lg_v, [(base + iota) * E + e])
                gt = ce > m1
                g2 = jnp.logical_and(jnp.logical_not(gt), ce > m2)
                m2n = jnp.where(gt, m1, jnp.where(g2, ce, m2))
                a2 = jnp.where(gt, a1, jnp.where(g2, e, a2))
                m2 = m2n
                a1 = jnp.where(gt, e, a1)
                m1 = jnp.where(gt, ce, m1)
            w0 = 1.0 / (1.0 + jnp.exp(m2 - m1))
            e0_v[pl.ds(base, L)] = a1
            e1_v[pl.ds(base, L)] = a2
            w0_v[pl.ds(base, L)] = w0
            w1_v[pl.ds(base, L)] = 1.0 - w0
            return carry

        lax.fori_loop(0, 128 // L, chunk, 0)
        pltpu.sync_copy(w0_v, w_hbm.at[pl.ds(s * 128, 128)])
        pltpu.sync_copy(w1_v, w_hbm.at[pl.ds(T + s * 128, 128)])
        pltpu.sync_copy(e0_v, ids_sp.at[pl.ds(s * 128, 128)])
        pltpu.sync_copy(e1_v, ids_sp.at[pl.ds(T + s * 128, 128)])

    plsc.subcore_barrier()

    @pl.when(on0)
    def _phase_b():
        pltpu.sync_copy(ids_sp, allids_v)

        def cb(i, cnt):
            v = allids_v[pl.ds(i * L, L)]
            return cnt + (v == s).astype(jnp.int32)

        cnt = lax.fori_loop(0, P // L, cb, jnp.zeros((L,), jnp.int32))
        tot = jnp.sum(cnt)
        c16_v[...] = jnp.full((L,), tot, jnp.int32)
        pltpu.sync_copy(c16_v, cnt_sp.at[s])

    plsc.subcore_barrier()

    @pl.when(on0)
    def _phase_c():
        pltpu.sync_copy(cnt_sp, cnts_v)
        counts = plsc.load_gather(cnts_v, [iota, iota])
        padded = ((counts + (BLK - 1)) >> 7) << 7
        cs = plsc.cumsum(padded)
        start = cs - padded

        def z1(i, carry):
            locsrc_v[pl.ds(i * L, L)] = jnp.zeros((L,), jnp.int32)
            return carry

        def z2(i, carry):
            locslot_v[pl.ds(i * L, L)] = jnp.zeros((L,), jnp.int32)
            return carry

        lax.fori_loop(0, NPAD // L, z1, 0)
        lax.fori_loop(0, P // L, z2, 0)

        start_s = _splat(start, s)

        def ce(i, rank):
            v = allids_v[pl.ds(i * L, L)]
            m = v == s
            mi = m.astype(jnp.int32)
            pcs = plsc.cumsum(mi)
            slots = rank + pcs - mi
            pairpos = i * L + iota
            tok = jnp.bitwise_and(pairpos, T - 1)
            plsc.store_scatter(locslot_v, [pairpos], slots, mask=m)
            plsc.store_scatter(locsrc_v, [slots], tok + 1, mask=m)
            return rank + _splat(pcs, L - 1)

        lax.fori_loop(0, P // L, ce, start_s)

        @pl.when(s == 0)
        def _bexp():
            startblk = start >> 7
            ne = (padded > 0).astype(jnp.int32)
            rank = plsc.cumsum(ne) - ne
            nv_v[...] = jnp.zeros((L,), jnp.int32)
            plsc.store_scatter(nv_v, [rank], iota, mask=padded > 0)
            nv2_v[...] = rank
            usedblk = jnp.sum(padded) >> 7
            nr_v[...] = jnp.where(
                iota == 0, jnp.full((L,), jnp.sum(ne), jnp.int32),
                jnp.where(iota == 1, jnp.full((L,), usedblk, jnp.int32), 0))
            for cc in range(NBLK // L):
                bvec = iota + cc * L
                acc = jnp.zeros((L,), jnp.int32)
                for e in range(E):
                    sb = _splat(startblk, e)
                    pe = _splat(padded, e)
                    cond = jnp.logical_and(sb <= bvec, pe > 0)
                    acc = jnp.where(cond, e, acc)
                be_v[pl.ds(cc * L, L)] = acc
                nx_v[pl.ds(cc * L, L)] = plsc.load_gather(nv2_v, [acc])
            pltpu.sync_copy(be_v, bexp_hbm)
            pltpu.sync_copy(nx_v, runof_hbm)
            pltpu.sync_copy(nv_v, runexp_hbm)
            pltpu.sync_copy(nr_v, nruns_hbm)

        pltpu.sync_copy(locsrc_v, stsrc_sp.at[s])
        pltpu.sync_copy(locslot_v, stslot_sp.at[s])

    plsc.subcore_barrier()

    SRCW = NPAD // L
    SLTW = P // L

    @pl.when(on0)
    def _phase_d():
        pltpu.sync_copy(stsrc_sp.at[:, pl.ds(s * SRCW, SRCW)], stga_v)
        pltpu.sync_copy(stslot_sp.at[:, pl.ds(s * SLTW, SLTW)], stgb_v)

        def addj(j, carry):
            acc = stga_v[0, pl.ds(j * L, L)]
            for r in range(1, L):
                acc = acc + stga_v[r, pl.ds(j * L, L)]
            ramp = jnp.bitwise_and(s * SRCW + j * L + iota, T - 1)
            stga_v[0, pl.ds(j * L, L)] = jnp.where(acc > 0, acc - 1, ramp)
            return carry

        lax.fori_loop(0, SRCW // L, addj, 0)
        pltpu.sync_copy(stga_v.at[0], src_hbm.at[pl.ds(s * SRCW, SRCW)])

        def addj2(j, carry):
            acc = stgb_v[0, pl.ds(j * L, L)]
            for r in range(1, L):
                acc = acc + stgb_v[r, pl.ds(j * L, L)]
            stgb_v[0, pl.ds(j * L, L)] = acc
            return carry

        lax.fori_loop(0, SLTW // L, addj2, 0)
        pltpu.sync_copy(stgb_v.at[0], slot_hbm.at[pl.ds(s * SLTW, SLTW)])


def _k2(logits_flat):
    mesh = plsc.VectorSubcoreMesh(core_axis_name="c", subcore_axis_name="s")
    f = pl.kernel(
        _k2_body,
        compiler_params=pltpu.CompilerParams(needs_layout_passes=False),
        out_type=[
            jax.ShapeDtypeStruct((NPAD,), jnp.int32),
            jax.ShapeDtypeStruct((P,), jnp.int32),
            jax.ShapeDtypeStruct((P,), jnp.float32),
            jax.ShapeDtypeStruct((NBLK,), jnp.int32),
            jax.ShapeDtypeStruct((NBLK,), jnp.int32),
            jax.ShapeDtypeStruct((L,), jnp.int32),
            jax.ShapeDtypeStruct((L,), jnp.int32),
        ],
        mesh=mesh,
        scratch_types=[
            pltpu.VMEM((128 * E,), jnp.float32),
            pltpu.VMEM((128,), jnp.int32),
            pltpu.VMEM((128,), jnp.int32),
            pltpu.VMEM((128,), jnp.float32),
            pltpu.VMEM((128,), jnp.float32),
            pltpu.VMEM((P,), jnp.int32),
            pltpu.VMEM((L, L), jnp.int32),
            pltpu.VMEM((L,), jnp.int32),
            pltpu.VMEM((NPAD,), jnp.int32),
            pltpu.VMEM((P,), jnp.int32),
            pltpu.VMEM((L, NPAD // L), jnp.int32),
            pltpu.VMEM((L, P // L), jnp.int32),
            pltpu.VMEM((NBLK,), jnp.int32),
            pltpu.VMEM((NBLK,), jnp.int32),
            pltpu.VMEM((L,), jnp.int32),
            pltpu.VMEM((L,), jnp.int32),
            pltpu.VMEM((L,), jnp.int32),
            pltpu.VMEM_SHARED((P,), jnp.int32),
            pltpu.VMEM_SHARED((L, L), jnp.int32),
            pltpu.VMEM_SHARED((L, NPAD), jnp.int32),
            pltpu.VMEM_SHARED((L, P), jnp.int32),
        ],
    )
    return f(logits_flat)


GCH = 32
GNCH = 6
GNBUF = 4


def _k2b_body(src_hbm, hf_hbm, xs_hbm, idx_v, r0, r1, r2, r3,
              g0, g1, g2, g3, w0, w1, w2, w3):
    c = lax.axis_index("c")
    s = lax.axis_index("s")
    wid = s * 2 + c
    per = NPAD // 32
    base = wid * per
    bufs = (r0, r1, r2, r3)
    gsems = (g0, g1, g2, g3)
    wsems = (w0, w1, w2, w3)
    pltpu.sync_copy(src_hbm.at[pl.ds(base, per)], idx_v)
    gcp = [None] * GNCH
    wcp = [None] * GNCH
    for j in range(GNCH):
        b = j % GNBUF
        if j >= GNBUF:
            wcp[j - GNBUF].wait()
        gcp[j] = pltpu.async_copy(
            hf_hbm.at[idx_v.at[pl.ds(j * GCH, GCH)]], bufs[b], gsems[b])
        if j >= 1:
            bp = (j - 1) % GNBUF
            gcp[j - 1].wait()
            wcp[j - 1] = pltpu.async_copy(
                bufs[bp], xs_hbm.at[pl.ds(base + (j - 1) * GCH, GCH), :],
                wsems[bp])
    gcp[GNCH - 1].wait()
    wcp[GNCH - 1] = pltpu.async_copy(
        bufs[(GNCH - 1) % GNBUF],
        xs_hbm.at[pl.ds(base + (GNCH - 1) * GCH, GCH), :],
        wsems[(GNCH - 1) % GNBUF])
    for j in range(GNCH - GNBUF, GNCH):
        if j >= 0 and wcp[j] is not None:
            wcp[j].wait()


def _k2b(src_idx, hf):
    mesh = plsc.VectorSubcoreMesh(core_axis_name="c", subcore_axis_name="s")
    f = pl.kernel(
        _k2b_body,
        compiler_params=pltpu.CompilerParams(needs_layout_passes=False),
        out_type=jax.ShapeDtypeStruct((NPAD, D), jnp.float32),
        mesh=mesh,
        scratch_types=[
            pltpu.VMEM((NPAD // 32,), jnp.int32),
            pltpu.VMEM((GCH, D), jnp.float32),
            pltpu.VMEM((GCH, D), jnp.float32),
            pltpu.VMEM((GCH, D), jnp.float32),
            pltpu.VMEM((GCH, D), jnp.float32),
            pltpu.SemaphoreType.DMA,
            pltpu.SemaphoreType.DMA,
            pltpu.SemaphoreType.DMA,
            pltpu.SemaphoreType.DMA,
            pltpu.SemaphoreType.DMA,
            pltpu.SemaphoreType.DMA,
            pltpu.SemaphoreType.DMA,
            pltpu.SemaphoreType.DMA,
        ],
    )
    return f(src_idx, hf)


NSLOT = 3


def _k3_body(be_ref, runof_ref, runexp_ref, nruns_ref, x_ref,
             w1_hbm, w2_hbm, w3_hbm, o_ref, w1b, w2b, w3b, s1, s2, s3):
    b = pl.program_id(0)
    nruns = nruns_ref[0]

    def fetch(e, slot):
        pltpu.async_copy(w1_hbm.at[e], w1b.at[slot], s1.at[slot])
        pltpu.async_copy(w2_hbm.at[e], w2b.at[slot], s2.at[slot])
        pltpu.async_copy(w3_hbm.at[e], w3b.at[slot], s3.at[slot])

    def wait(slot):
        pltpu.make_async_copy(w1_hbm.at[0], w1b.at[slot], s1.at[slot]).wait()
        pltpu.make_async_copy(w2_hbm.at[0], w2b.at[slot], s2.at[slot]).wait()
        pltpu.make_async_copy(w3_hbm.at[0], w3b.at[slot], s3.at[slot]).wait()

    rc = runof_ref[b]

    @pl.when(b == 0)
    def _init():
        for r in range(NSLOT):
            @pl.when(r < nruns)
            def _():
                fetch(runexp_ref[r], r)

        wait(0)

    changed = jnp.logical_and(b > 0, rc != runof_ref[jnp.maximum(b - 1, 0)])

    @pl.when(changed)
    def _advance():
        pr = rc + NSLOT - 1

        @pl.when(pr < nruns)
        def _():
            fetch(runexp_ref[pr], lax.rem(pr, NSLOT))

        wait(lax.rem(rc, NSLOT))

    @pl.when(b < nruns_ref[1])
    def _compute():
        slot = lax.rem(rc, NSLOT)
        x = x_ref[...]
        z1 = _dot_nt(x, w1b[slot])
        z3 = _dot_nt(x, w3b[slot])
        act = z1 * jax.nn.sigmoid(z1) * z3
        o_ref[...] = _dot_nt(act, w2b[slot])


def _k3(block_expert, runof, runexp, nruns, xs, W1, W2, W3):
    grid_spec = pltpu.PrefetchScalarGridSpec(
        num_scalar_prefetch=4,
        grid=(NBLK,),
        in_specs=[
            pl.BlockSpec((BLK, D), lambda b, be, ro, re, nr: (b, 0)),
            pl.BlockSpec(memory_space=pl.ANY),
            pl.BlockSpec(memory_space=pl.ANY),
            pl.BlockSpec(memory_space=pl.ANY),
        ],
        out_specs=pl.BlockSpec((BLK, D), lambda b, be, ro, re, nr: (b, 0)),
        scratch_shapes=[
            pltpu.VMEM((NSLOT, D, D), jnp.float32),
            pltpu.VMEM((NSLOT, D, D), jnp.float32),
            pltpu.VMEM((NSLOT, D, D), jnp.float32),
            pltpu.SemaphoreType.DMA((NSLOT,)),
            pltpu.SemaphoreType.DMA((NSLOT,)),
            pltpu.SemaphoreType.DMA((NSLOT,)),
        ],
    )
    return pl.pallas_call(
        _k3_body,
        grid_spec=grid_spec,
        out_shape=jax.ShapeDtypeStruct((NPAD, D), jnp.float32),
    )(block_expert, runof, runexp, nruns, xs, W1, W2, W3)


CCH = 32


def _k4_body(outs_hbm, sh_hbm, slot_hbm, w_hbm, y_hbm,
             idx0_v, idx1_v, w0_v, w1_v, r0_v, r1_v, shv_v, y_v,
             sem, sem2, sem3, ysem):
    c = lax.axis_index("c")
    s = lax.axis_index("s")
    wid = s * 2 + c
    per = T // 32
    iota = lax.iota(jnp.int32, L)
    ycp = None
    for jc in range(per // CCH):
        t0 = wid * per + jc * CCH
        pltpu.sync_copy(slot_hbm.at[pl.ds(t0, CCH)], idx0_v)
        pltpu.sync_copy(slot_hbm.at[pl.ds(T + t0, CCH)], idx1_v)
        pltpu.sync_copy(w_hbm.at[pl.ds(t0, CCH)], w0_v)
        pltpu.sync_copy(w_hbm.at[pl.ds(T + t0, CCH)], w1_v)
        cp0 = pltpu.async_copy(outs_hbm.at[idx0_v], r0_v, sem)
        cp1 = pltpu.async_copy(outs_hbm.at[idx1_v], r1_v, sem2)
        cps = pltpu.async_copy(sh_hbm.at[pl.ds(t0, CCH), :], shv_v, sem3)
        cp0.wait()
        cp1.wait()
        cps.wait()
        if ycp is not None:
            ycp.wait()

        def tok(i, carry):
            g = i >> 4
            lane = jnp.bitwise_and(i, L - 1)
            w0g = w0_v[pl.ds(g * L, L)]
            w1g = w1_v[pl.ds(g * L, L)]
            w0s = jnp.full((L,), jnp.sum(jnp.where(iota == lane, w0g, 0.0)), jnp.float32)
            w1s = jnp.full((L,), jnp.sum(jnp.where(iota == lane, w1g, 0.0)), jnp.float32)

            def col(jj, carry2):
                sl = pl.ds(jj * L, L)
                y_v[i, sl] = (r0_v[i, sl] * w0s + r1_v[i, sl] * w1s
                              + shv_v[i, sl])
                return carry2

            lax.fori_loop(0, D // L, col, 0)
            return carry

        lax.fori_loop(0, CCH, tok, 0)
        ycp = pltpu.async_copy(y_v, y_hbm.at[pl.ds(t0, CCH), :], ysem)
    ycp.wait()


def _k4(outs, shared_y, slot_flat, w_flat):
    mesh = plsc.VectorSubcoreMesh(core_axis_name="c", subcore_axis_name="s")
    f = pl.kernel(
        _k4_body,
        compiler_params=pltpu.CompilerParams(needs_layout_passes=False),
        out_type=jax.ShapeDtypeStruct((T, D), jnp.float32),
        mesh=mesh,
        scratch_types=[
            pltpu.VMEM((CCH,), jnp.int32),
            pltpu.VMEM((CCH,), jnp.int32),
            pltpu.VMEM((CCH,), jnp.float32),
            pltpu.VMEM((CCH,), jnp.float32),
            pltpu.VMEM((CCH, D), jnp.float32),
            pltpu.VMEM((CCH, D), jnp.float32),
            pltpu.VMEM((CCH, D), jnp.float32),
            pltpu.VMEM((CCH, D), jnp.float32),
            pltpu.SemaphoreType.DMA,
            pltpu.SemaphoreType.DMA,
            pltpu.SemaphoreType.DMA,
            pltpu.SemaphoreType.DMA,
        ],
    )
    return f(outs, shared_y, slot_flat, w_flat)


def kernel(h, Wg, W1, W2, W3, W1s, W2s, W3s):
    b, s, d = h.shape
    hf = h.reshape(T, D)
    logits = _k1a(hf, Wg)
    shared_y = _k1b(hf, W1s, W2s, W3s)
    src_idx, slot_flat, w_flat, block_expert, runof, runexp, nruns = _k2(
        logits.reshape(-1))
    xs = _k2b(src_idx, hf)
    outs = _k3(block_expert, runof, runexp, nruns, xs, W1, W2, W3)
    y = _k4(outs, shared_y, slot_flat, w_flat)
    return y.reshape(b, s, d)
